# baseline (speedup 1.0000x reference)
import jax
import jax.numpy as jnp
from jax import lax
from jax.experimental import pallas as pl
from jax.experimental.pallas import tpu as pltpu
from jax.experimental.pallas import tpu_sc as plsc

N = 10000
D = 128
E = 320000
NPAD = 10240
NW = 32
CH = 128
CPT = 80
HPT = CPT // 2
EPT = CPT * CH
EPAD = NW * EPT
RPT = NPAD // 16
BR = 1024
BE = 2048
ZPOS = 50.0


def _pe_table():
    inv_freq = 1.0 / (55 * 10) ** (jnp.arange(0, D, 2, dtype=jnp.float32) / D)
    t = jnp.arange(0, 55, dtype=jnp.float32)[:, None]
    ang = t * inv_freq[None, :]
    pe = jnp.concatenate([jnp.sin(ang), jnp.cos(ang)], axis=1)
    return jnp.pad(pe, ((0, 64 - 55), (0, 0)))


def _deg_body(dstc_hbm, zer_hbm, one_hbm, out_hbm, di2, ones_v, rows, ssem,
              deg_sh):
    c = lax.axis_index("c")
    s = lax.axis_index("s")
    wid = c * 16 + s
    pltpu.sync_copy(zer_hbm, deg_sh.at[pl.ds(s * RPT, RPT)])
    pltpu.sync_copy(dstc_hbm.at[pl.ds(wid * CPT, CPT)], di2)
    pltpu.sync_copy(one_hbm, ones_v)
    plsc.subcore_barrier()

    def sca(j, b):
        return pltpu.make_async_copy(ones_v, deg_sh.at[di2.at[j]],
                                     ssem.at[b])

    def chunk(j, carry):
        b = lax.rem(j, 2)

        @pl.when(j >= 2)
        def _():
            sca(j - 2, b).wait()

        sca(j, b).start(add=True)
        return carry

    lax.fori_loop(0, CPT, chunk, 0)
    for j in (CPT - 2, CPT - 1):
        sca(j, j % 2).wait()
    plsc.subcore_barrier()

    bufs = [rows, ones_v]
    nrb = RPT // CH
    for k in range(nrb):
        b = bufs[k % 2]

        def wrd(kk, bb):
            return pltpu.make_async_copy(
                bb, out_hbm.at[pl.ds(c * NPAD + s * RPT + kk * CH, CH)],
                ssem.at[kk % 2])

        if k >= 2:
            wrd(k - 2, b).wait()
        pltpu.sync_copy(deg_sh.at[pl.ds(s * RPT + k * CH, CH)], b)
        wrd(k, b).start()
    for k in (nrb - 2, nrb - 1):
        pltpu.make_async_copy(
            bufs[k % 2],
            out_hbm.at[pl.ds(c * NPAD + s * RPT + k * CH, CH)],
            ssem.at[k % 2]).wait()


def _conv_body(y_hbm, srcc_hbm, dstc_hbm, zer_hbm, out_hbm, si2, di2, rows,
               gsem, ssem, acc_sh):
    c = lax.axis_index("c")
    s = lax.axis_index("s")
    wid = c * 16 + s
    pltpu.sync_copy(zer_hbm, acc_sh.at[pl.ds(s * RPT, RPT)])
    plsc.subcore_barrier()

    def gat(j, b):
        return pltpu.make_async_copy(y_hbm.at[si2.at[j]], rows.at[b],
                                     gsem.at[b])

    def sca(j, b):
        return pltpu.make_async_copy(rows.at[b], acc_sh.at[di2.at[j]],
                                     ssem.at[b])

    for h in range(CPT // HPT):
        cb = wid * CPT + h * HPT
        pltpu.sync_copy(srcc_hbm.at[pl.ds(cb, HPT)], si2)
        pltpu.sync_copy(dstc_hbm.at[pl.ds(cb, HPT)], di2)
        gat(0, 0).start()

        def chunk(j, carry):
            b = lax.rem(j, 2)
            b2 = 1 - b

            @pl.when(j >= 1)
            def _():
                sca(j - 1, b2).wait()

            @pl.when(j + 1 < HPT)
            def _():
                gat(j + 1, b2).start()

            gat(j, b).wait()
            sca(j, b).start(add=True)
            return carry

        lax.fori_loop(0, HPT, chunk, 0)
        sca(HPT - 1, (HPT - 1) % 2).wait()
    plsc.subcore_barrier()

    def rb(k, carry):
        b = lax.rem(k, 2)
        r = s * RPT + k * CH

        @pl.when(k >= 2)
        def _():
            pltpu.make_async_copy(
                rows.at[b], out_hbm.at[pl.ds(c * NPAD + (k - 2) * CH
                                             + s * RPT, CH)],
                ssem.at[b]).wait()

        pltpu.sync_copy(acc_sh.at[pl.ds(r, CH)], rows.at[b])
        pltpu.make_async_copy(rows.at[b],
                              out_hbm.at[pl.ds(c * NPAD + r, CH)],
                              ssem.at[b]).start()
        return carry

    nrb = RPT // CH
    lax.fori_loop(0, nrb, rb, 0)
    for k in (nrb - 2, nrb - 1):
        pltpu.make_async_copy(
            rows.at[k % 2],
            out_hbm.at[pl.ds(c * NPAD + s * RPT + k * CH, CH)],
            ssem.at[k % 2]).wait()


def _edge_body(xs_hbm, xd_hbm, srcc_hbm, dstc_hbm, out_hbm, si2, di2, rows,
               gsem, asem, wsem):
    c = lax.axis_index("c")
    s = lax.axis_index("s")
    wid = c * 16 + s
    pltpu.sync_copy(srcc_hbm.at[pl.ds(wid * CPT, CPT)], si2)
    pltpu.sync_copy(dstc_hbm.at[pl.ds(wid * CPT, CPT)], di2)

    def gat(j, b):
        return pltpu.make_async_copy(xs_hbm.at[si2.at[j]], rows.at[b],
                                     gsem.at[b])

    def gadd(j, b):
        return pltpu.make_async_copy(xd_hbm.at[di2.at[j]], rows.at[b],
                                     asem.at[b])

    def wr(j, b):
        return pltpu.make_async_copy(
            rows.at[b], out_hbm.at[pl.ds((wid * CPT + j) * CH, CH)],
            wsem.at[b])

    gat(0, 0).start()

    def chunk(j, carry):
        b = lax.rem(j, 2)
        b2 = 1 - b

        @pl.when(j >= 1)
        def _():
            wr(j - 1, b2).wait()

        @pl.when(j + 1 < CPT)
        def _():
            gat(j + 1, b2).start()

        gat(j, b).wait()
        gadd(j, b).start(add=True)
        gadd(j, b).wait()
        wr(j, b).start()
        return carry

    lax.fori_loop(0, CPT, chunk, 0)
    wr(CPT - 1, (CPT - 1) % 2).wait()


def _a_body(inp_ref, degp_ref, pe_ref, a1w_ref, a1b_ref, a2w_ref, a2b_ref,
            c1w_ref, y1_ref, dis_ref):
    xb = inp_ref[...]
    t = jnp.maximum(jnp.dot(xb, a1w_ref[...],
                            preferred_element_type=jnp.float32) + a1b_ref[...],
                    0.0)
    f = jnp.dot(t, a2w_ref[...], preferred_element_type=jnp.float32) \
        + a2b_ref[...]
    pos = (xb[:, 0:1] * ZPOS).astype(jnp.int32)
    iot = lax.broadcasted_iota(jnp.int32, (1, 64), 1)
    oh = (pos == iot).astype(jnp.float32)
    x0 = f + jnp.dot(oh, pe_ref[...], preferred_element_type=jnp.float32)
    dp = degp_ref[...]
    deg = dp[0, :, 0:1] + dp[1, :, 0:1] + 1.0
    dis = lax.rsqrt(deg)
    dis_ref[...] = dis
    y1_ref[...] = dis * jnp.dot(x0, c1w_ref[...],
                                preferred_element_type=jnp.float32)


def _c_body(accp_ref, y_ref, dis_ref, b_ref, w_ref, yout_ref):
    ap = accp_ref[...]
    dis = dis_ref[...]
    x = jnp.maximum(dis * (ap[0] + ap[1] + y_ref[...]) + b_ref[...], 0.0)
    yout_ref[...] = dis * jnp.dot(x, w_ref[...],
                                  preferred_element_type=jnp.float32)


def _c4_body(accp_ref, y_ref, dis_ref, inp4_ref, c3b_ref, clsw1_ref,
             clsb1_ref, clsw2_ref, clsb2_ref, boxw1_ref, boxb1_ref, la_ref,
             lb_ref, ew1a_ref, ew1b_ref, eb1_ref, x3_ref, pred_ref, box_ref,
             xs_ref, xd_ref):
    ap = accp_ref[...]
    dis = dis_ref[...]
    x3 = dis * (ap[0] + ap[1] + y_ref[...]) + c3b_ref[...]
    x3_ref[...] = x3
    p = jnp.maximum(jnp.dot(x3, clsw1_ref[...],
                            preferred_element_type=jnp.float32)
                    + clsb1_ref[...], 0.0)
    pred_ref[...] = jnp.dot(p, clsw2_ref[...],
                            preferred_element_type=jnp.float32) + clsb2_ref[...]
    h = jnp.maximum(jnp.dot(x3, boxw1_ref[...],
                            preferred_element_type=jnp.float32)
                    + boxb1_ref[...], 0.0)
    ha = jnp.dot(h, la_ref[...], preferred_element_type=jnp.float32)
    hb = jnp.dot(ha, lb_ref[...], preferred_element_type=jnp.float32)
    box_ref[...] = jnp.tanh(hb[:, 0:4]) + inp4_ref[...]
    xs_ref[...] = jnp.dot(x3, ew1a_ref[...],
                          preferred_element_type=jnp.float32) + eb1_ref[...]
    xd_ref[...] = jnp.dot(x3, ew1b_ref[...],
                          preferred_element_type=jnp.float32)


def _f_body(h_ref, w2_ref, b2_ref, w3_ref, b3_ref, o_ref):
    h = jnp.maximum(h_ref[...], 0.0)
    h = jnp.maximum(jnp.dot(h, w2_ref[...],
                            preferred_element_type=jnp.float32) + b2_ref[...],
                    0.0)
    o_ref[...] = jax.nn.sigmoid(
        jnp.dot(h, w3_ref[...], preferred_element_type=jnp.float32)
        + b3_ref[...])


def _full(shape):
    return pl.BlockSpec(shape, lambda i: tuple(0 for _ in shape))


def kernel(inputs, edge_index, a1_W, a1_b, a2_W, a2_b, c1_W, c1_b, c2_W, c2_b,
           c3_W, c3_b, cls_W1, cls_b1, cls_W2, cls_b2, box_W1, box_b1, lora_A,
           lora_B, e_W1, e_b1, e_W2, e_b2, e_W3, e_b3):
    f32 = jnp.float32
    pe = _pe_table()
    inp_p = jnp.pad(inputs, ((0, NPAD - N), (0, 0)))
    src_p = jnp.pad(edge_index[0], (0, EPAD - E), constant_values=N)
    dst_p = jnp.pad(edge_index[1], (0, EPAD - E), constant_values=N)
    srcc = src_p.reshape(NW * CPT, CH)
    dstc = dst_p.reshape(NW * CPT, CH)
    zer_d = jnp.zeros((RPT, D), f32)
    one_d = jnp.ones((CH, D), f32)

    mesh = plsc.VectorSubcoreMesh(core_axis_name="c", subcore_axis_name="s")

    deg_call = pl.kernel(
        _deg_body,
        out_type=jax.ShapeDtypeStruct((2 * NPAD, D), f32),
        mesh=mesh,
        scratch_types=[
            pltpu.VMEM((CPT, CH), jnp.int32),
            pltpu.VMEM((CH, D), f32),
            pltpu.VMEM((CH, D), f32),
            pltpu.SemaphoreType.DMA((2,)),
            pltpu.VMEM_SHARED((NPAD, D), f32),
        ],
    )
    degp = deg_call(dstc, zer_d, one_d).reshape(2, NPAD, D)

    grid = NPAD // BR
    y1, dis = pl.pallas_call(
        _a_body,
        grid=(grid,),
        in_specs=[
            pl.BlockSpec((BR, D), lambda i: (i, 0)),
            pl.BlockSpec((2, BR, D), lambda i: (0, i, 0)),
            _full((64, D)), _full((D, D)), _full((1, D)),
            _full((D, D)), _full((1, D)), _full((D, D)),
        ],
        out_specs=[pl.BlockSpec((BR, D), lambda i: (i, 0)),
                   pl.BlockSpec((BR, 1), lambda i: (i, 0))],
        out_shape=[jax.ShapeDtypeStruct((NPAD, D), f32),
                   jax.ShapeDtypeStruct((NPAD, 1), f32)],
    )(inp_p, degp, pe, a1_W, a1_b.reshape(1, D), a2_W, a2_b.reshape(1, D),
      c1_W)

    conv_call = pl.kernel(
        _conv_body,
        out_type=jax.ShapeDtypeStruct((2 * NPAD, D), f32),
        mesh=mesh,
        scratch_types=[
            pltpu.VMEM((HPT, CH), jnp.int32),
            pltpu.VMEM((HPT, CH), jnp.int32),
            pltpu.VMEM((2, CH, D), f32),
            pltpu.SemaphoreType.DMA((2,)),
            pltpu.SemaphoreType.DMA((2,)),
            pltpu.VMEM_SHARED((NPAD, D), f32),
        ],
    )

    def conv_epilogue(accp, y, b, w):
        return pl.pallas_call(
            _c_body,
            grid=(grid,),
            in_specs=[
                pl.BlockSpec((2, BR, D), lambda i: (0, i, 0)),
                pl.BlockSpec((BR, D), lambda i: (i, 0)),
                pl.BlockSpec((BR, 1), lambda i: (i, 0)),
                _full((1, D)), _full((D, D)),
            ],
            out_specs=pl.BlockSpec((BR, D), lambda i: (i, 0)),
            out_shape=jax.ShapeDtypeStruct((NPAD, D), f32),
        )(accp, y, dis, b.reshape(1, D), w)

    accp1 = conv_call(y1, srcc, dstc, zer_d).reshape(2, NPAD, D)
    y2 = conv_epilogue(accp1, y1, c1_b, c2_W)
    accp2 = conv_call(y2, srcc, dstc, zer_d).reshape(2, NPAD, D)
    y3 = conv_epilogue(accp2, y2, c2_b, c3_W)
    accp3 = conv_call(y3, srcc, dstc, zer_d).reshape(2, NPAD, D)

    lap = jnp.pad(lora_A, ((0, 0), (0, 4)))
    lbp = jnp.pad(lora_B, ((0, 4), (0, 4)))
    x3, pred, box, xs1, xd1 = pl.pallas_call(
        _c4_body,
        grid=(grid,),
        in_specs=[
            pl.BlockSpec((2, BR, D), lambda i: (0, i, 0)),
            pl.BlockSpec((BR, D), lambda i: (i, 0)),
            pl.BlockSpec((BR, 1), lambda i: (i, 0)),
            pl.BlockSpec((BR, 4), lambda i: (i, 0)),
            _full((1, D)),
            _full((D, D // 2)), _full((1, D // 2)),
            _full((D // 2, 16)), _full((1, 16)),
            _full((D, D // 2)), _full((1, D // 2)),
            _full((D // 2, 8)), _full((8, 8)),
            _full((D, D)), _full((D, D)), _full((1, D)),
        ],
        out_specs=[pl.BlockSpec((BR, D), lambda i: (i, 0)),
                   pl.BlockSpec((BR, 16), lambda i: (i, 0)),
                   pl.BlockSpec((BR, 4), lambda i: (i, 0)),
                   pl.BlockSpec((BR, D), lambda i: (i, 0)),
                   pl.BlockSpec((BR, D), lambda i: (i, 0))],
        out_shape=[jax.ShapeDtypeStruct((NPAD, D), f32),
                   jax.ShapeDtypeStruct((NPAD, 16), f32),
                   jax.ShapeDtypeStruct((NPAD, 4), f32),
                   jax.ShapeDtypeStruct((NPAD, D), f32),
                   jax.ShapeDtypeStruct((NPAD, D), f32)],
    )(accp3, y3, dis, inp_p[:, 1:5], c3_b.reshape(1, D), cls_W1,
      cls_b1.reshape(1, D // 2), cls_W2, cls_b2.reshape(1, 16), box_W1,
      box_b1.reshape(1, D // 2), lap, lbp, e_W1[:D], e_W1[D:],
      e_b1.reshape(1, D))

    edge_call = pl.kernel(
        _edge_body,
        out_type=jax.ShapeDtypeStruct((EPAD, D), f32),
        mesh=mesh,
        scratch_types=[
            pltpu.VMEM((CPT, CH), jnp.int32),
            pltpu.VMEM((CPT, CH), jnp.int32),
            pltpu.VMEM((2, CH, D), f32),
            pltpu.SemaphoreType.DMA((2,)),
            pltpu.SemaphoreType.DMA((2,)),
            pltpu.SemaphoreType.DMA((2,)),
        ],
    )
    h1 = edge_call(xs1, xd1, srcc, dstc)

    edge_full = pl.pallas_call(
        _f_body,
        grid=(EPAD // BE,),
        in_specs=[
            pl.BlockSpec((BE, D), lambda i: (i, 0)),
            _full((D, D // 2)), _full((1, D // 2)),
            _full((D // 2, 8)), _full((1, 8)),
        ],
        out_specs=pl.BlockSpec((BE, 8), lambda i: (i, 0)),
        out_shape=jax.ShapeDtypeStruct((EPAD, 8), f32),
    )(h1, e_W2, e_b2.reshape(1, D // 2), jnp.pad(e_W3, ((0, 0), (0, 7))),
      jnp.pad(e_b3.reshape(1, 1), ((0, 0), (0, 7))))

    return (pred[:N], box[:N], edge_full[:E, 0:1], x3[:N])


# speedup vs baseline: 1.0561x; 1.0561x over previous
import jax
import jax.numpy as jnp
from jax import lax
from jax.experimental import pallas as pl
from jax.experimental.pallas import tpu as pltpu
from jax.experimental.pallas import tpu_sc as plsc

N = 10000
D = 128
E = 320000
NPAD = 10240
NW = 32
CH = 128
CPT = 80
CPT0 = 130
CPT1 = 30
TCH = NW * CPT
EPT = CPT * CH
EPAD = NW * EPT
RPT = NPAD // 16
BR = 1024
BE = 2048
ZPOS = 50.0


def _pe_table():
    inv_freq = 1.0 / (55 * 10) ** (jnp.arange(0, D, 2, dtype=jnp.float32) / D)
    t = jnp.arange(0, 55, dtype=jnp.float32)[:, None]
    ang = t * inv_freq[None, :]
    pe = jnp.concatenate([jnp.sin(ang), jnp.cos(ang)], axis=1)
    return jnp.pad(pe, ((0, 64 - 55), (0, 0)))


def _deg_body(dstc_hbm, zer_hbm, one_hbm, out_hbm, di2, ones_v, rows, ssem,
              deg_sh):
    c = lax.axis_index("c")
    s = lax.axis_index("s")
    wid = c * 16 + s
    pltpu.sync_copy(zer_hbm, deg_sh.at[pl.ds(s * RPT, RPT)])
    pltpu.sync_copy(dstc_hbm.at[pl.ds(wid * CPT, CPT)], di2)
    pltpu.sync_copy(one_hbm, ones_v)
    plsc.subcore_barrier()

    def sca(j, b):
        return pltpu.make_async_copy(ones_v, deg_sh.at[di2.at[j]],
                                     ssem.at[b])

    def chunk(j, carry):
        b = lax.rem(j, 2)

        @pl.when(j >= 2)
        def _():
            sca(j - 2, b).wait()

        sca(j, b).start(add=True)
        return carry

    lax.fori_loop(0, CPT, chunk, 0)
    for j in (CPT - 2, CPT - 1):
        sca(j, j % 2).wait()
    plsc.subcore_barrier()

    bufs = [rows, ones_v]
    nrb = RPT // CH
    for k in range(nrb):
        b = bufs[k % 2]

        def wrd(kk, bb):
            return pltpu.make_async_copy(
                bb, out_hbm.at[pl.ds(c * NPAD + s * RPT + kk * CH, CH)],
                ssem.at[kk % 2])

        if k >= 2:
            wrd(k - 2, b).wait()
        pltpu.sync_copy(deg_sh.at[pl.ds(s * RPT + k * CH, CH)], b)
        wrd(k, b).start()
    for k in (nrb - 2, nrb - 1):
        pltpu.make_async_copy(
            bufs[k % 2],
            out_hbm.at[pl.ds(c * NPAD + s * RPT + k * CH, CH)],
            ssem.at[k % 2]).wait()


def _conv_body(y_hbm, srcc_hbm, dstc_hbm, zer_hbm, out_hbm, si, di, rows,
               sism, dism, gsem, ssem, acc_sh):
    c = lax.axis_index("c")
    s = lax.axis_index("s")
    cpt = jnp.where(c == 0, CPT0, CPT1)
    cb = jnp.where(c == 0, s * CPT0, 16 * CPT0 + s * CPT1)
    pltpu.sync_copy(zer_hbm, acc_sh.at[pl.ds(s * RPT, RPT)])
    plsc.subcore_barrier()

    def ils(j):
        bi = lax.rem(j, 3)
        return (pltpu.make_async_copy(srcc_hbm.at[cb + j], si.at[bi],
                                      sism.at[bi]),
                pltpu.make_async_copy(dstc_hbm.at[cb + j], di.at[bi],
                                      dism.at[bi]))

    def gat(j):
        b = lax.rem(j, 2)
        return pltpu.make_async_copy(y_hbm.at[si.at[lax.rem(j, 3)]],
                                     rows.at[b], gsem.at[b])

    def sca(j):
        b = lax.rem(j, 2)
        return pltpu.make_async_copy(rows.at[b],
                                     acc_sh.at[di.at[lax.rem(j, 3)]],
                                     ssem.at[b])

    for d in ils(0) + ils(1):
        d.start()
    for d in ils(0):
        d.wait()
    gat(0).start()

    def chunk(j, carry):
        @pl.when(j >= 1)
        def _():
            sca(j - 1).wait()

        @pl.when(j + 2 < cpt)
        def _():
            for d in ils(j + 2):
                d.start()

        @pl.when(j + 1 < cpt)
        def _():
            for d in ils(j + 1):
                d.wait()
            gat(j + 1).start()

        gat(j).wait()
        sca(j).start(add=True)
        return carry

    lax.fori_loop(0, cpt, chunk, 0)
    sca(cpt - 1).wait()
    plsc.subcore_barrier()

    def rb(k, carry):
        b = lax.rem(k, 2)
        r = s * RPT + k * CH

        @pl.when(k >= 2)
        def _():
            pltpu.make_async_copy(
                rows.at[b], out_hbm.at[pl.ds(c * NPAD + (k - 2) * CH
                                             + s * RPT, CH)],
                ssem.at[b]).wait()

        pltpu.sync_copy(acc_sh.at[pl.ds(r, CH)], rows.at[b])
        pltpu.make_async_copy(rows.at[b],
                              out_hbm.at[pl.ds(c * NPAD + r, CH)],
                              ssem.at[b]).start()
        return carry

    nrb = RPT // CH
    lax.fori_loop(0, nrb, rb, 0)
    for k in (nrb - 2, nrb - 1):
        pltpu.make_async_copy(
            rows.at[k % 2],
            out_hbm.at[pl.ds(c * NPAD + s * RPT + k * CH, CH)],
            ssem.at[k % 2]).wait()


def _edge_body(xs_hbm, xd_hbm, srcc_hbm, dstc_hbm, out_hbm, si, di, rows,
               sism, dism, gsem, asem, wsem):
    c = lax.axis_index("c")
    s = lax.axis_index("s")
    cpt = jnp.where(c == 0, CPT0, CPT1)
    cb = jnp.where(c == 0, s * CPT0, 16 * CPT0 + s * CPT1)

    def ils(j):
        bi = lax.rem(j, 3)
        return (pltpu.make_async_copy(srcc_hbm.at[cb + j], si.at[bi],
                                      sism.at[bi]),
                pltpu.make_async_copy(dstc_hbm.at[cb + j], di.at[bi],
                                      dism.at[bi]))

    def gat(j):
        b = lax.rem(j, 2)
        return pltpu.make_async_copy(xs_hbm.at[si.at[lax.rem(j, 3)]],
                                     rows.at[b], gsem.at[b])

    def gadd(j):
        b = lax.rem(j, 2)
        return pltpu.make_async_copy(xd_hbm.at[di.at[lax.rem(j, 3)]],
                                     rows.at[b], asem.at[b])

    def wr(j):
        b = lax.rem(j, 2)
        return pltpu.make_async_copy(
            rows.at[b], out_hbm.at[pl.ds((cb + j) * CH, CH)], wsem.at[b])

    for d in ils(0) + ils(1):
        d.start()
    for d in ils(0):
        d.wait()
    gat(0).start()

    def chunk(j, carry):
        @pl.when(j >= 1)
        def _():
            wr(j - 1).wait()

        @pl.when(j + 2 < cpt)
        def _():
            for d in ils(j + 2):
                d.start()

        @pl.when(j + 1 < cpt)
        def _():
            for d in ils(j + 1):
                d.wait()
            gat(j + 1).start()

        gat(j).wait()
        gadd(j).start(add=True)
        gadd(j).wait()
        wr(j).start()
        return carry

    lax.fori_loop(0, cpt, chunk, 0)
    wr(cpt - 1).wait()


def _a_body(inp_ref, degp_ref, pe_ref, a1w_ref, a1b_ref, a2w_ref, a2b_ref,
            c1w_ref, y1_ref, dis_ref):
    xb = inp_ref[...]
    t = jnp.maximum(jnp.dot(xb, a1w_ref[...],
                            preferred_element_type=jnp.float32) + a1b_ref[...],
                    0.0)
    f = jnp.dot(t, a2w_ref[...], preferred_element_type=jnp.float32) \
        + a2b_ref[...]
    pos = (xb[:, 0:1] * ZPOS).astype(jnp.int32)
    iot = lax.broadcasted_iota(jnp.int32, (1, 64), 1)
    oh = (pos == iot).astype(jnp.float32)
    x0 = f + jnp.dot(oh, pe_ref[...], preferred_element_type=jnp.float32)
    dp = degp_ref[...]
    deg = dp[0, :, 0:1] + dp[1, :, 0:1] + 1.0
    dis = lax.rsqrt(deg)
    dis_ref[...] = dis
    y1_ref[...] = dis * jnp.dot(x0, c1w_ref[...],
                                preferred_element_type=jnp.float32)


def _c_body(accp_ref, y_ref, dis_ref, b_ref, w_ref, yout_ref):
    ap = accp_ref[...]
    dis = dis_ref[...]
    x = jnp.maximum(dis * (ap[0] + ap[1] + y_ref[...]) + b_ref[...], 0.0)
    yout_ref[...] = dis * jnp.dot(x, w_ref[...],
                                  preferred_element_type=jnp.float32)


def _c4_body(accp_ref, y_ref, dis_ref, inp4_ref, c3b_ref, clsw1_ref,
             clsb1_ref, clsw2_ref, clsb2_ref, boxw1_ref, boxb1_ref, la_ref,
             lb_ref, ew1a_ref, ew1b_ref, eb1_ref, x3_ref, pred_ref, box_ref,
             xs_ref, xd_ref):
    ap = accp_ref[...]
    dis = dis_ref[...]
    x3 = dis * (ap[0] + ap[1] + y_ref[...]) + c3b_ref[...]
    x3_ref[...] = x3
    p = jnp.maximum(jnp.dot(x3, clsw1_ref[...],
                            preferred_element_type=jnp.float32)
                    + clsb1_ref[...], 0.0)
    pred_ref[...] = jnp.dot(p, clsw2_ref[...],
                            preferred_element_type=jnp.float32) + clsb2_ref[...]
    h = jnp.maximum(jnp.dot(x3, boxw1_ref[...],
                            preferred_element_type=jnp.float32)
                    + boxb1_ref[...], 0.0)
    ha = jnp.dot(h, la_ref[...], preferred_element_type=jnp.float32)
    hb = jnp.dot(ha, lb_ref[...], preferred_element_type=jnp.float32)
    box_ref[...] = jnp.tanh(hb[:, 0:4]) + inp4_ref[...]
    xs_ref[...] = jnp.dot(x3, ew1a_ref[...],
                          preferred_element_type=jnp.float32) + eb1_ref[...]
    xd_ref[...] = jnp.dot(x3, ew1b_ref[...],
                          preferred_element_type=jnp.float32)


def _f_body(h_ref, w2_ref, b2_ref, w3_ref, b3_ref, o_ref):
    h = jnp.maximum(h_ref[...], 0.0)
    h = jnp.maximum(jnp.dot(h, w2_ref[...],
                            preferred_element_type=jnp.float32) + b2_ref[...],
                    0.0)
    o_ref[...] = jax.nn.sigmoid(
        jnp.dot(h, w3_ref[...], preferred_element_type=jnp.float32)
        + b3_ref[...])


def _full(shape):
    return pl.BlockSpec(shape, lambda i: tuple(0 for _ in shape))


def _deg_scratch():
    f32 = jnp.float32
    return [
        pltpu.VMEM((CPT, CH), jnp.int32),
        pltpu.VMEM((CH, D), f32),
        pltpu.VMEM((CH, D), f32),
        pltpu.SemaphoreType.DMA((2,)),
        pltpu.VMEM_SHARED((NPAD, D), f32),
    ]


def _conv_scratch():
    f32 = jnp.float32
    return [
        pltpu.VMEM((3, CH), jnp.int32),
        pltpu.VMEM((3, CH), jnp.int32),
        pltpu.VMEM((2, CH, D), f32),
        pltpu.SemaphoreType.DMA((3,)),
        pltpu.SemaphoreType.DMA((3,)),
        pltpu.SemaphoreType.DMA((2,)),
        pltpu.SemaphoreType.DMA((2,)),
        pltpu.VMEM_SHARED((NPAD, D), f32),
    ]


def _edge_scratch():
    f32 = jnp.float32
    return [
        pltpu.VMEM((3, CH), jnp.int32),
        pltpu.VMEM((3, CH), jnp.int32),
        pltpu.VMEM((2, CH, D), f32),
        pltpu.SemaphoreType.DMA((3,)),
        pltpu.SemaphoreType.DMA((3,)),
        pltpu.SemaphoreType.DMA((2,)),
        pltpu.SemaphoreType.DMA((2,)),
        pltpu.SemaphoreType.DMA((2,)),
    ]


def kernel(inputs, edge_index, a1_W, a1_b, a2_W, a2_b, c1_W, c1_b, c2_W, c2_b,
           c3_W, c3_b, cls_W1, cls_b1, cls_W2, cls_b2, box_W1, box_b1, lora_A,
           lora_B, e_W1, e_b1, e_W2, e_b2, e_W3, e_b3):
    f32 = jnp.float32
    pe = _pe_table()
    inp_p = jnp.pad(inputs, ((0, NPAD - N), (0, 0)))
    src_p = jnp.pad(edge_index[0], (0, EPAD - E), constant_values=N)
    dst_p = jnp.pad(edge_index[1], (0, EPAD - E), constant_values=N)
    srcc = src_p.reshape(NW * CPT, CH)
    dstc = dst_p.reshape(NW * CPT, CH)
    zer_d = jnp.zeros((RPT, D), f32)
    one_d = jnp.ones((CH, D), f32)

    mesh = plsc.VectorSubcoreMesh(core_axis_name="c", subcore_axis_name="s")

    deg_call = pl.kernel(
        _deg_body,
        out_type=jax.ShapeDtypeStruct((2 * NPAD, D), f32),
        mesh=mesh,
        scratch_types=_deg_scratch(),
    )
    degp = deg_call(dstc, zer_d, one_d).reshape(2, NPAD, D)

    grid = NPAD // BR
    y1, dis = pl.pallas_call(
        _a_body,
        grid=(grid,),
        in_specs=[
            pl.BlockSpec((BR, D), lambda i: (i, 0)),
            pl.BlockSpec((2, BR, D), lambda i: (0, i, 0)),
            _full((64, D)), _full((D, D)), _full((1, D)),
            _full((D, D)), _full((1, D)), _full((D, D)),
        ],
        out_specs=[pl.BlockSpec((BR, D), lambda i: (i, 0)),
                   pl.BlockSpec((BR, 1), lambda i: (i, 0))],
        out_shape=[jax.ShapeDtypeStruct((NPAD, D), f32),
                   jax.ShapeDtypeStruct((NPAD, 1), f32)],
    )(inp_p, degp, pe, a1_W, a1_b.reshape(1, D), a2_W, a2_b.reshape(1, D),
      c1_W)

    conv_call = pl.kernel(
        _conv_body,
        out_type=jax.ShapeDtypeStruct((2 * NPAD, D), f32),
        mesh=mesh,
        scratch_types=_conv_scratch(),
    )

    def conv_epilogue(accp, y, b, w):
        return pl.pallas_call(
            _c_body,
            grid=(grid,),
            in_specs=[
                pl.BlockSpec((2, BR, D), lambda i: (0, i, 0)),
                pl.BlockSpec((BR, D), lambda i: (i, 0)),
                pl.BlockSpec((BR, 1), lambda i: (i, 0)),
                _full((1, D)), _full((D, D)),
            ],
            out_specs=pl.BlockSpec((BR, D), lambda i: (i, 0)),
            out_shape=jax.ShapeDtypeStruct((NPAD, D), f32),
        )(accp, y, dis, b.reshape(1, D), w)

    accp1 = conv_call(y1, srcc, dstc, zer_d).reshape(2, NPAD, D)
    y2 = conv_epilogue(accp1, y1, c1_b, c2_W)
    accp2 = conv_call(y2, srcc, dstc, zer_d).reshape(2, NPAD, D)
    y3 = conv_epilogue(accp2, y2, c2_b, c3_W)
    accp3 = conv_call(y3, srcc, dstc, zer_d).reshape(2, NPAD, D)

    lap = jnp.pad(lora_A, ((0, 0), (0, 4)))
    lbp = jnp.pad(lora_B, ((0, 4), (0, 4)))
    x3, pred, box, xs1, xd1 = pl.pallas_call(
        _c4_body,
        grid=(grid,),
        in_specs=[
            pl.BlockSpec((2, BR, D), lambda i: (0, i, 0)),
            pl.BlockSpec((BR, D), lambda i: (i, 0)),
            pl.BlockSpec((BR, 1), lambda i: (i, 0)),
            pl.BlockSpec((BR, 4), lambda i: (i, 0)),
            _full((1, D)),
            _full((D, D // 2)), _full((1, D // 2)),
            _full((D // 2, 16)), _full((1, 16)),
            _full((D, D // 2)), _full((1, D // 2)),
            _full((D // 2, 8)), _full((8, 8)),
            _full((D, D)), _full((D, D)), _full((1, D)),
        ],
        out_specs=[pl.BlockSpec((BR, D), lambda i: (i, 0)),
                   pl.BlockSpec((BR, 16), lambda i: (i, 0)),
                   pl.BlockSpec((BR, 4), lambda i: (i, 0)),
                   pl.BlockSpec((BR, D), lambda i: (i, 0)),
                   pl.BlockSpec((BR, D), lambda i: (i, 0))],
        out_shape=[jax.ShapeDtypeStruct((NPAD, D), f32),
                   jax.ShapeDtypeStruct((NPAD, 16), f32),
                   jax.ShapeDtypeStruct((NPAD, 4), f32),
                   jax.ShapeDtypeStruct((NPAD, D), f32),
                   jax.ShapeDtypeStruct((NPAD, D), f32)],
    )(accp3, y3, dis, inp_p[:, 1:5], c3_b.reshape(1, D), cls_W1,
      cls_b1.reshape(1, D // 2), cls_W2, cls_b2.reshape(1, 16), box_W1,
      box_b1.reshape(1, D // 2), lap, lbp, e_W1[:D], e_W1[D:],
      e_b1.reshape(1, D))

    edge_call = pl.kernel(
        _edge_body,
        out_type=jax.ShapeDtypeStruct((EPAD, D), f32),
        mesh=mesh,
        scratch_types=_edge_scratch(),
    )
    h1 = edge_call(xs1, xd1, srcc, dstc)

    edge_full = pl.pallas_call(
        _f_body,
        grid=(EPAD // BE,),
        in_specs=[
            pl.BlockSpec((BE, D), lambda i: (i, 0)),
            _full((D, D // 2)), _full((1, D // 2)),
            _full((D // 2, 8)), _full((1, 8)),
        ],
        out_specs=pl.BlockSpec((BE, 8), lambda i: (i, 0)),
        out_shape=jax.ShapeDtypeStruct((EPAD, 8), f32),
    )(h1, e_W2, e_b2.reshape(1, D // 2), jnp.pad(e_W3, ((0, 0), (0, 7))),
      jnp.pad(e_b3.reshape(1, 1), ((0, 0), (0, 7))))

    return (pred[:N], box[:N], edge_full[:E, 0:1], x3[:N])


# speedup vs baseline: 2.8328x; 2.6824x over previous
import jax
import jax.numpy as jnp
from jax import lax
from jax.experimental import pallas as pl
from jax.experimental.pallas import tpu as pltpu
from jax.experimental.pallas import tpu_sc as plsc

N = 10000
D = 128
E = 320000
NPAD = 10240
NW = 32
CH = 128
CPT = 80
CPT0 = 80
CPT1 = 80
TCH = NW * CPT
EPT = CPT * CH
EPAD = NW * EPT
RPT = NPAD // 16
BR = 1024
BE = 2048
ZPOS = 50.0


def _pe_table():
    inv_freq = 1.0 / (55 * 10) ** (jnp.arange(0, D, 2, dtype=jnp.float32) / D)
    t = jnp.arange(0, 55, dtype=jnp.float32)[:, None]
    ang = t * inv_freq[None, :]
    pe = jnp.concatenate([jnp.sin(ang), jnp.cos(ang)], axis=1)
    return jnp.pad(pe, ((0, 64 - 55), (0, 0)))


def _deg_body(dstc_hbm, zer_hbm, one_hbm, out_hbm, di2, ones_v, rows, ssem,
              deg_sh):
    c = lax.axis_index("c")
    s = lax.axis_index("s")
    wid = c * 16 + s
    pltpu.sync_copy(zer_hbm, deg_sh.at[pl.ds(s * RPT, RPT)])
    pltpu.sync_copy(dstc_hbm.at[pl.ds(wid * CPT, CPT)], di2)
    pltpu.sync_copy(one_hbm, ones_v)
    plsc.subcore_barrier()

    def sca(j, b):
        return pltpu.make_async_copy(ones_v, deg_sh.at[di2.at[j]],
                                     ssem.at[b])

    def chunk(j, carry):
        b = lax.rem(j, 2)

        @pl.when(j >= 2)
        def _():
            sca(j - 2, b).wait()

        sca(j, b).start(add=True)
        return carry

    lax.fori_loop(0, CPT, chunk, 0)
    for j in (CPT - 2, CPT - 1):
        sca(j, j % 2).wait()
    plsc.subcore_barrier()

    bufs = [rows, ones_v]
    nrb = RPT // CH
    for k in range(nrb):
        b = bufs[k % 2]

        def wrd(kk, bb):
            return pltpu.make_async_copy(
                bb, out_hbm.at[pl.ds(c * NPAD + s * RPT + kk * CH, CH)],
                ssem.at[kk % 2])

        if k >= 2:
            wrd(k - 2, b).wait()
        pltpu.sync_copy(deg_sh.at[pl.ds(s * RPT + k * CH, CH)], b)
        wrd(k, b).start()
    for k in (nrb - 2, nrb - 1):
        pltpu.make_async_copy(
            bufs[k % 2],
            out_hbm.at[pl.ds(c * NPAD + s * RPT + k * CH, CH)],
            ssem.at[k % 2]).wait()


def _conv_body(y_hbm, srcc_hbm, dstc_hbm, zer_hbm, out_hbm, si, di, rows,
               sism, dism, gsem, ssem, acc_sh):
    c = lax.axis_index("c")
    s = lax.axis_index("s")
    cpt = jnp.where(c == 0, CPT0, CPT1)
    cb = jnp.where(c == 0, s * CPT0, 16 * CPT0 + s * CPT1)
    pltpu.sync_copy(zer_hbm, acc_sh.at[pl.ds(s * RPT, RPT)])
    plsc.subcore_barrier()

    def ils(j):
        bi = lax.rem(j, 3)
        return (pltpu.make_async_copy(srcc_hbm.at[cb + j], si.at[bi],
                                      sism.at[bi]),
                pltpu.make_async_copy(dstc_hbm.at[cb + j], di.at[bi],
                                      dism.at[bi]))

    def gat(j):
        b = lax.rem(j, 2)
        return pltpu.make_async_copy(y_hbm.at[si.at[lax.rem(j, 3)]],
                                     rows.at[b], gsem.at[b])

    def sca(j):
        b = lax.rem(j, 2)
        return pltpu.make_async_copy(rows.at[b],
                                     acc_sh.at[di.at[lax.rem(j, 3)]],
                                     ssem.at[b])

    for d in ils(0) + ils(1):
        d.start()
    for d in ils(0):
        d.wait()
    gat(0).start()

    def chunk(j, carry):
        @pl.when(j >= 1)
        def _():
            sca(j - 1).wait()

        @pl.when(j + 2 < cpt)
        def _():
            for d in ils(j + 2):
                d.start()

        @pl.when(j + 1 < cpt)
        def _():
            for d in ils(j + 1):
                d.wait()
            gat(j + 1).start()

        gat(j).wait()
        sca(j).start(add=True)
        return carry

    lax.fori_loop(0, cpt, chunk, 0)
    sca(cpt - 1).wait()
    plsc.subcore_barrier()

    def rb(k, carry):
        b = lax.rem(k, 2)
        r = s * RPT + k * CH

        @pl.when(k >= 2)
        def _():
            pltpu.make_async_copy(
                rows.at[b], out_hbm.at[pl.ds(c * NPAD + (k - 2) * CH
                                             + s * RPT, CH)],
                ssem.at[b]).wait()

        pltpu.sync_copy(acc_sh.at[pl.ds(r, CH)], rows.at[b])
        pltpu.make_async_copy(rows.at[b],
                              out_hbm.at[pl.ds(c * NPAD + r, CH)],
                              ssem.at[b]).start()
        return carry

    nrb = RPT // CH
    lax.fori_loop(0, nrb, rb, 0)
    for k in (nrb - 2, nrb - 1):
        pltpu.make_async_copy(
            rows.at[k % 2],
            out_hbm.at[pl.ds(c * NPAD + s * RPT + k * CH, CH)],
            ssem.at[k % 2]).wait()


def _edge_body(xs_hbm, xd_hbm, srcc_hbm, dstc_hbm, out_hbm, si, di, rows,
               sism, dism, gsem, asem, wsem):
    c = lax.axis_index("c")
    s = lax.axis_index("s")
    cpt = jnp.where(c == 0, CPT0, CPT1)
    cb = jnp.where(c == 0, s * CPT0, 16 * CPT0 + s * CPT1)

    def ils(j):
        bi = lax.rem(j, 3)
        return (pltpu.make_async_copy(srcc_hbm.at[cb + j], si.at[bi],
                                      sism.at[bi]),
                pltpu.make_async_copy(dstc_hbm.at[cb + j], di.at[bi],
                                      dism.at[bi]))

    def gat(j):
        b = lax.rem(j, 2)
        return pltpu.make_async_copy(xs_hbm.at[si.at[lax.rem(j, 3)]],
                                     rows.at[b], gsem.at[b])

    def gadd(j):
        b = lax.rem(j, 2)
        return pltpu.make_async_copy(xd_hbm.at[di.at[lax.rem(j, 3)]],
                                     rows.at[b], asem.at[b])

    def wr(j):
        b = lax.rem(j, 2)
        return pltpu.make_async_copy(
            rows.at[b], out_hbm.at[pl.ds((cb + j) * CH, CH)], wsem.at[b])

    for d in ils(0) + ils(1):
        d.start()
    for d in ils(0):
        d.wait()
    gat(0).start()

    def chunk(j, carry):
        @pl.when(j >= 1)
        def _():
            wr(j - 1).wait()

        @pl.when(j + 2 < cpt)
        def _():
            for d in ils(j + 2):
                d.start()

        @pl.when(j + 1 < cpt)
        def _():
            for d in ils(j + 1):
                d.wait()
            gat(j + 1).start()

        gat(j).wait()
        gadd(j).start(add=True)
        gadd(j).wait()
        wr(j).start()
        return carry

    lax.fori_loop(0, cpt, chunk, 0)
    wr(cpt - 1).wait()


def _a_body(inp_ref, degp_ref, pe_ref, a1w_ref, a1b_ref, a2w_ref, a2b_ref,
            c1w_ref, y1_ref, dis_ref):
    xb = inp_ref[...]
    t = jnp.maximum(jnp.dot(xb, a1w_ref[...],
                            preferred_element_type=jnp.float32) + a1b_ref[...],
                    0.0)
    f = jnp.dot(t, a2w_ref[...], preferred_element_type=jnp.float32) \
        + a2b_ref[...]
    pos = (xb[:, 0:1] * ZPOS).astype(jnp.int32)
    iot = lax.broadcasted_iota(jnp.int32, (1, 64), 1)
    oh = (pos == iot).astype(jnp.float32)
    x0 = f + jnp.dot(oh, pe_ref[...], preferred_element_type=jnp.float32)
    dp = degp_ref[...]
    deg = dp[0, :, 0:1] + dp[1, :, 0:1] + 1.0
    dis = lax.rsqrt(deg)
    dis_ref[...] = dis
    y1_ref[...] = dis * jnp.dot(x0, c1w_ref[...],
                                preferred_element_type=jnp.float32)


def _c_body(accp_ref, y_ref, dis_ref, b_ref, w_ref, yout_ref):
    ap = accp_ref[...]
    dis = dis_ref[...]
    x = jnp.maximum(dis * (ap[0] + ap[1] + y_ref[...]) + b_ref[...], 0.0)
    yout_ref[...] = dis * jnp.dot(x, w_ref[...],
                                  preferred_element_type=jnp.float32)


def _c4_body(accp_ref, y_ref, dis_ref, inp4_ref, c3b_ref, clsw1_ref,
             clsb1_ref, clsw2_ref, clsb2_ref, boxw1_ref, boxb1_ref, la_ref,
             lb_ref, ew1a_ref, ew1b_ref, eb1_ref, x3_ref, pred_ref, box_ref,
             xs_ref, xd_ref):
    ap = accp_ref[...]
    dis = dis_ref[...]
    x3 = dis * (ap[0] + ap[1] + y_ref[...]) + c3b_ref[...]
    x3_ref[...] = x3
    p = jnp.maximum(jnp.dot(x3, clsw1_ref[...],
                            preferred_element_type=jnp.float32)
                    + clsb1_ref[...], 0.0)
    pred_ref[...] = jnp.dot(p, clsw2_ref[...],
                            preferred_element_type=jnp.float32) + clsb2_ref[...]
    h = jnp.maximum(jnp.dot(x3, boxw1_ref[...],
                            preferred_element_type=jnp.float32)
                    + boxb1_ref[...], 0.0)
    ha = jnp.dot(h, la_ref[...], preferred_element_type=jnp.float32)
    hb = jnp.dot(ha, lb_ref[...], preferred_element_type=jnp.float32)
    box_ref[...] = jnp.tanh(hb[:, 0:4]) + inp4_ref[...]
    xs_ref[...] = jnp.dot(x3, ew1a_ref[...],
                          preferred_element_type=jnp.float32) + eb1_ref[...]
    xd_ref[...] = jnp.dot(x3, ew1b_ref[...],
                          preferred_element_type=jnp.float32)


def _f_body(h_ref, w2_ref, b2_ref, w3_ref, b3_ref, o_ref):
    h = jnp.maximum(h_ref[...], 0.0)
    h = jnp.maximum(jnp.dot(h, w2_ref[...],
                            preferred_element_type=jnp.float32) + b2_ref[...],
                    0.0)
    o_ref[...] = jax.nn.sigmoid(
        jnp.dot(h, w3_ref[...], preferred_element_type=jnp.float32)
        + b3_ref[...])


def _full(shape):
    return pl.BlockSpec(shape, lambda i: tuple(0 for _ in shape))


def _deg_scratch():
    f32 = jnp.float32
    return [
        pltpu.VMEM((CPT, CH), jnp.int32),
        pltpu.VMEM((CH, D), f32),
        pltpu.VMEM((CH, D), f32),
        pltpu.SemaphoreType.DMA((2,)),
        pltpu.VMEM_SHARED((NPAD, D), f32),
    ]


def _conv_scratch():
    f32 = jnp.float32
    return [
        pltpu.VMEM((3, CH), jnp.int32),
        pltpu.VMEM((3, CH), jnp.int32),
        pltpu.VMEM((2, CH, D), f32),
        pltpu.SemaphoreType.DMA((3,)),
        pltpu.SemaphoreType.DMA((3,)),
        pltpu.SemaphoreType.DMA((2,)),
        pltpu.SemaphoreType.DMA((2,)),
        pltpu.VMEM_SHARED((NPAD, D), f32),
    ]


def _edge_scratch():
    f32 = jnp.float32
    return [
        pltpu.VMEM((3, CH), jnp.int32),
        pltpu.VMEM((3, CH), jnp.int32),
        pltpu.VMEM((2, CH, D), f32),
        pltpu.SemaphoreType.DMA((3,)),
        pltpu.SemaphoreType.DMA((3,)),
        pltpu.SemaphoreType.DMA((2,)),
        pltpu.SemaphoreType.DMA((2,)),
        pltpu.SemaphoreType.DMA((2,)),
    ]


def kernel(inputs, edge_index, a1_W, a1_b, a2_W, a2_b, c1_W, c1_b, c2_W, c2_b,
           c3_W, c3_b, cls_W1, cls_b1, cls_W2, cls_b2, box_W1, box_b1, lora_A,
           lora_B, e_W1, e_b1, e_W2, e_b2, e_W3, e_b3):
    f32 = jnp.float32
    pe = _pe_table()
    inp_p = jnp.pad(inputs, ((0, NPAD - N), (0, 0)))
    pads = (N + jnp.arange(EPAD - E, dtype=jnp.int32) % (NPAD - N))
    src_p = jnp.concatenate([edge_index[0], pads])
    dst_p = jnp.concatenate([edge_index[1], pads])
    srcc = src_p.reshape(NW * CPT, CH)
    dstc = dst_p.reshape(NW * CPT, CH)
    zer_d = jnp.zeros((RPT, D), f32)
    one_d = jnp.ones((CH, D), f32)

    mesh = plsc.VectorSubcoreMesh(core_axis_name="c", subcore_axis_name="s")

    deg_call = pl.kernel(
        _deg_body,
        out_type=jax.ShapeDtypeStruct((2 * NPAD, D), f32),
        mesh=mesh,
        scratch_types=_deg_scratch(),
    )
    degp = deg_call(dstc, zer_d, one_d).reshape(2, NPAD, D)

    grid = NPAD // BR
    y1, dis = pl.pallas_call(
        _a_body,
        grid=(grid,),
        in_specs=[
            pl.BlockSpec((BR, D), lambda i: (i, 0)),
            pl.BlockSpec((2, BR, D), lambda i: (0, i, 0)),
            _full((64, D)), _full((D, D)), _full((1, D)),
            _full((D, D)), _full((1, D)), _full((D, D)),
        ],
        out_specs=[pl.BlockSpec((BR, D), lambda i: (i, 0)),
                   pl.BlockSpec((BR, 1), lambda i: (i, 0))],
        out_shape=[jax.ShapeDtypeStruct((NPAD, D), f32),
                   jax.ShapeDtypeStruct((NPAD, 1), f32)],
    )(inp_p, degp, pe, a1_W, a1_b.reshape(1, D), a2_W, a2_b.reshape(1, D),
      c1_W)

    conv_call = pl.kernel(
        _conv_body,
        out_type=jax.ShapeDtypeStruct((2 * NPAD, D), f32),
        mesh=mesh,
        scratch_types=_conv_scratch(),
    )

    def conv_epilogue(accp, y, b, w):
        return pl.pallas_call(
            _c_body,
            grid=(grid,),
            in_specs=[
                pl.BlockSpec((2, BR, D), lambda i: (0, i, 0)),
                pl.BlockSpec((BR, D), lambda i: (i, 0)),
                pl.BlockSpec((BR, 1), lambda i: (i, 0)),
                _full((1, D)), _full((D, D)),
            ],
            out_specs=pl.BlockSpec((BR, D), lambda i: (i, 0)),
            out_shape=jax.ShapeDtypeStruct((NPAD, D), f32),
        )(accp, y, dis, b.reshape(1, D), w)

    accp1 = conv_call(y1, srcc, dstc, zer_d).reshape(2, NPAD, D)
    y2 = conv_epilogue(accp1, y1, c1_b, c2_W)
    accp2 = conv_call(y2, srcc, dstc, zer_d).reshape(2, NPAD, D)
    y3 = conv_epilogue(accp2, y2, c2_b, c3_W)
    accp3 = conv_call(y3, srcc, dstc, zer_d).reshape(2, NPAD, D)

    lap = jnp.pad(lora_A, ((0, 0), (0, 4)))
    lbp = jnp.pad(lora_B, ((0, 4), (0, 4)))
    x3, pred, box, xs1, xd1 = pl.pallas_call(
        _c4_body,
        grid=(grid,),
        in_specs=[
            pl.BlockSpec((2, BR, D), lambda i: (0, i, 0)),
            pl.BlockSpec((BR, D), lambda i: (i, 0)),
            pl.BlockSpec((BR, 1), lambda i: (i, 0)),
            pl.BlockSpec((BR, 4), lambda i: (i, 0)),
            _full((1, D)),
            _full((D, D // 2)), _full((1, D // 2)),
            _full((D // 2, 16)), _full((1, 16)),
            _full((D, D // 2)), _full((1, D // 2)),
            _full((D // 2, 8)), _full((8, 8)),
            _full((D, D)), _full((D, D)), _full((1, D)),
        ],
        out_specs=[pl.BlockSpec((BR, D), lambda i: (i, 0)),
                   pl.BlockSpec((BR, 16), lambda i: (i, 0)),
                   pl.BlockSpec((BR, 4), lambda i: (i, 0)),
                   pl.BlockSpec((BR, D), lambda i: (i, 0)),
                   pl.BlockSpec((BR, D), lambda i: (i, 0))],
        out_shape=[jax.ShapeDtypeStruct((NPAD, D), f32),
                   jax.ShapeDtypeStruct((NPAD, 16), f32),
                   jax.ShapeDtypeStruct((NPAD, 4), f32),
                   jax.ShapeDtypeStruct((NPAD, D), f32),
                   jax.ShapeDtypeStruct((NPAD, D), f32)],
    )(accp3, y3, dis, inp_p[:, 1:5], c3_b.reshape(1, D), cls_W1,
      cls_b1.reshape(1, D // 2), cls_W2, cls_b2.reshape(1, 16), box_W1,
      box_b1.reshape(1, D // 2), lap, lbp, e_W1[:D], e_W1[D:],
      e_b1.reshape(1, D))

    edge_call = pl.kernel(
        _edge_body,
        out_type=jax.ShapeDtypeStruct((EPAD, D), f32),
        mesh=mesh,
        scratch_types=_edge_scratch(),
    )
    h1 = edge_call(xs1, xd1, srcc, dstc)

    edge_full = pl.pallas_call(
        _f_body,
        grid=(EPAD // BE,),
        in_specs=[
            pl.BlockSpec((BE, D), lambda i: (i, 0)),
            _full((D, D // 2)), _full((1, D // 2)),
            _full((D // 2, 8)), _full((1, 8)),
        ],
        out_specs=pl.BlockSpec((BE, 8), lambda i: (i, 0)),
        out_shape=jax.ShapeDtypeStruct((EPAD, 8), f32),
    )(h1, e_W2, e_b2.reshape(1, D // 2), jnp.pad(e_W3, ((0, 0), (0, 7))),
      jnp.pad(e_b3.reshape(1, 1), ((0, 0), (0, 7))))

    return (pred[:N], box[:N], edge_full[:E, 0:1], x3[:N])


# speedup vs baseline: 2.9787x; 1.0515x over previous
import jax
import jax.numpy as jnp
from jax import lax
from jax.experimental import pallas as pl
from jax.experimental.pallas import tpu as pltpu
from jax.experimental.pallas import tpu_sc as plsc

N = 10000
D = 128
E = 320000
NPAD = 10240
NW = 32
CH = 128
CPT = 80
CPT0 = 80
CPT1 = 80
TCH = NW * CPT
EPT = CPT * CH
EPAD = NW * EPT
RPT = NPAD // 16
BR = 1024
BE = 4096
ZPOS = 50.0


def _pe_table():
    inv_freq = 1.0 / (55 * 10) ** (jnp.arange(0, D, 2, dtype=jnp.float32) / D)
    t = jnp.arange(0, 55, dtype=jnp.float32)[:, None]
    ang = t * inv_freq[None, :]
    pe = jnp.concatenate([jnp.sin(ang), jnp.cos(ang)], axis=1)
    return jnp.pad(pe, ((0, 64 - 55), (0, 0)))


def _deg_body(dstc_hbm, zer_hbm, one_hbm, out_hbm, di2, ones_v, rows, ssem,
              deg_sh):
    c = lax.axis_index("c")
    s = lax.axis_index("s")
    wid = c * 16 + s
    pltpu.sync_copy(zer_hbm, deg_sh.at[pl.ds(s * RPT, RPT)])
    pltpu.sync_copy(dstc_hbm.at[pl.ds(wid * CPT, CPT)], di2)
    pltpu.sync_copy(one_hbm, ones_v)
    plsc.subcore_barrier()

    def sca(j, b):
        return pltpu.make_async_copy(ones_v, deg_sh.at[di2.at[j]],
                                     ssem.at[b])

    def chunk(j, carry):
        b = lax.rem(j, 2)

        @pl.when(j >= 2)
        def _():
            sca(j - 2, b).wait()

        sca(j, b).start(add=True)
        return carry

    lax.fori_loop(0, CPT, chunk, 0)
    for j in (CPT - 2, CPT - 1):
        sca(j, j % 2).wait()
    plsc.subcore_barrier()

    bufs = [rows, ones_v]
    nrb = RPT // CH
    for k in range(nrb):
        b = bufs[k % 2]

        def wrd(kk, bb):
            return pltpu.make_async_copy(
                bb, out_hbm.at[pl.ds(c * NPAD + s * RPT + kk * CH, CH)],
                ssem.at[kk % 2])

        if k >= 2:
            wrd(k - 2, b).wait()
        pltpu.sync_copy(deg_sh.at[pl.ds(s * RPT + k * CH, CH)], b)
        wrd(k, b).start()
    for k in (nrb - 2, nrb - 1):
        pltpu.make_async_copy(
            bufs[k % 2],
            out_hbm.at[pl.ds(c * NPAD + s * RPT + k * CH, CH)],
            ssem.at[k % 2]).wait()


def _conv_body(y_hbm, srcc_hbm, dstc_hbm, zer_hbm, out_hbm, si, di, rows,
               sism, dism, gsem, ssem, acc_sh):
    c = lax.axis_index("c")
    s = lax.axis_index("s")
    cpt = jnp.where(c == 0, CPT0, CPT1)
    cb = jnp.where(c == 0, s * CPT0, 16 * CPT0 + s * CPT1)
    pltpu.sync_copy(zer_hbm, acc_sh.at[pl.ds(s * RPT, RPT)])
    plsc.subcore_barrier()

    def ils(j):
        bi = lax.rem(j, 3)
        return (pltpu.make_async_copy(srcc_hbm.at[cb + j], si.at[bi],
                                      sism.at[bi]),
                pltpu.make_async_copy(dstc_hbm.at[cb + j], di.at[bi],
                                      dism.at[bi]))

    def gat(j):
        b = lax.rem(j, 2)
        return pltpu.make_async_copy(y_hbm.at[si.at[lax.rem(j, 3)]],
                                     rows.at[b], gsem.at[b])

    def sca(j):
        b = lax.rem(j, 2)
        return pltpu.make_async_copy(rows.at[b],
                                     acc_sh.at[di.at[lax.rem(j, 3)]],
                                     ssem.at[b])

    for d in ils(0) + ils(1):
        d.start()
    for d in ils(0):
        d.wait()
    gat(0).start()

    def chunk(j, carry):
        @pl.when(j >= 1)
        def _():
            sca(j - 1).wait()

        @pl.when(j + 2 < cpt)
        def _():
            for d in ils(j + 2):
                d.start()

        @pl.when(j + 1 < cpt)
        def _():
            for d in ils(j + 1):
                d.wait()
            gat(j + 1).start()

        gat(j).wait()
        sca(j).start(add=True)
        return carry

    lax.fori_loop(0, cpt, chunk, 0)
    sca(cpt - 1).wait()
    plsc.subcore_barrier()

    def rb(k, carry):
        b = lax.rem(k, 2)
        r = s * RPT + k * CH

        @pl.when(k >= 2)
        def _():
            pltpu.make_async_copy(
                rows.at[b], out_hbm.at[pl.ds(c * NPAD + (k - 2) * CH
                                             + s * RPT, CH)],
                ssem.at[b]).wait()

        pltpu.sync_copy(acc_sh.at[pl.ds(r, CH)], rows.at[b])
        pltpu.make_async_copy(rows.at[b],
                              out_hbm.at[pl.ds(c * NPAD + r, CH)],
                              ssem.at[b]).start()
        return carry

    nrb = RPT // CH
    lax.fori_loop(0, nrb, rb, 0)
    for k in (nrb - 2, nrb - 1):
        pltpu.make_async_copy(
            rows.at[k % 2],
            out_hbm.at[pl.ds(c * NPAD + s * RPT + k * CH, CH)],
            ssem.at[k % 2]).wait()


def _edge_body(xs_hbm, xd_hbm, srcc_hbm, dstc_hbm, out_hbm, si, di, rows,
               sism, dism, gsem, asem, wsem):
    c = lax.axis_index("c")
    s = lax.axis_index("s")
    cpt = jnp.where(c == 0, CPT0, CPT1)
    cb = jnp.where(c == 0, s * CPT0, 16 * CPT0 + s * CPT1)

    def ils(j):
        bi = lax.rem(j, 3)
        return (pltpu.make_async_copy(srcc_hbm.at[cb + j], si.at[bi],
                                      sism.at[bi]),
                pltpu.make_async_copy(dstc_hbm.at[cb + j], di.at[bi],
                                      dism.at[bi]))

    def gat(j):
        b = lax.rem(j, 2)
        return pltpu.make_async_copy(xs_hbm.at[si.at[lax.rem(j, 3)]],
                                     rows.at[b], gsem.at[b])

    def gadd(j):
        b = lax.rem(j, 2)
        return pltpu.make_async_copy(xd_hbm.at[di.at[lax.rem(j, 3)]],
                                     rows.at[b], asem.at[b])

    def wr(j):
        b = lax.rem(j, 2)
        return pltpu.make_async_copy(
            rows.at[b], out_hbm.at[pl.ds((cb + j) * CH, CH)], wsem.at[b])

    for d in ils(0) + ils(1):
        d.start()
    for d in ils(0):
        d.wait()
    gat(0).start()

    def chunk(j, carry):
        @pl.when(j >= 1)
        def _():
            wr(j - 1).wait()

        @pl.when(j + 2 < cpt)
        def _():
            for d in ils(j + 2):
                d.start()

        @pl.when(j + 1 < cpt)
        def _():
            for d in ils(j + 1):
                d.wait()
            gat(j + 1).start()

        gat(j).wait()
        gadd(j).start(add=True)
        gadd(j).wait()
        wr(j).start()
        return carry

    lax.fori_loop(0, cpt, chunk, 0)
    wr(cpt - 1).wait()


def _a_body(inp_ref, degp_ref, pe_ref, a1w_ref, a1b_ref, a2w_ref, a2b_ref,
            c1w_ref, y1_ref, dis_ref):
    xb = inp_ref[...]
    t = jnp.maximum(jnp.dot(xb, a1w_ref[...],
                            preferred_element_type=jnp.float32) + a1b_ref[...],
                    0.0)
    f = jnp.dot(t, a2w_ref[...], preferred_element_type=jnp.float32) \
        + a2b_ref[...]
    pos = (xb[:, 0:1] * ZPOS).astype(jnp.int32)
    iot = lax.broadcasted_iota(jnp.int32, (1, 64), 1)
    oh = (pos == iot).astype(jnp.float32)
    x0 = f + jnp.dot(oh, pe_ref[...], preferred_element_type=jnp.float32)
    dp = degp_ref[...]
    deg = dp[0, :, 0:1] + dp[1, :, 0:1] + 1.0
    dis = lax.rsqrt(deg)
    dis_ref[...] = dis
    y1_ref[...] = dis * jnp.dot(x0, c1w_ref[...],
                                preferred_element_type=jnp.float32)


def _c_body(accp_ref, y_ref, dis_ref, b_ref, w_ref, yout_ref):
    ap = accp_ref[...]
    dis = dis_ref[...]
    x = jnp.maximum(dis * (ap[0] + ap[1] + y_ref[...]) + b_ref[...], 0.0)
    yout_ref[...] = dis * jnp.dot(x, w_ref[...],
                                  preferred_element_type=jnp.float32)


def _c4_body(accp_ref, y_ref, dis_ref, inp4_ref, c3b_ref, clsw1_ref,
             clsb1_ref, clsw2_ref, clsb2_ref, boxw1_ref, boxb1_ref, la_ref,
             lb_ref, ew1a_ref, ew1b_ref, eb1_ref, x3_ref, pred_ref, box_ref,
             xs_ref, xd_ref):
    ap = accp_ref[...]
    dis = dis_ref[...]
    x3 = dis * (ap[0] + ap[1] + y_ref[...]) + c3b_ref[...]
    x3_ref[...] = x3
    p = jnp.maximum(jnp.dot(x3, clsw1_ref[...],
                            preferred_element_type=jnp.float32)
                    + clsb1_ref[...], 0.0)
    pred_ref[...] = jnp.dot(p, clsw2_ref[...],
                            preferred_element_type=jnp.float32) + clsb2_ref[...]
    h = jnp.maximum(jnp.dot(x3, boxw1_ref[...],
                            preferred_element_type=jnp.float32)
                    + boxb1_ref[...], 0.0)
    ha = jnp.dot(h, la_ref[...], preferred_element_type=jnp.float32)
    hb = jnp.dot(ha, lb_ref[...], preferred_element_type=jnp.float32)
    box_ref[...] = jnp.tanh(hb[:, 0:4]) + inp4_ref[...]
    xs_ref[...] = jnp.dot(x3, ew1a_ref[...],
                          preferred_element_type=jnp.float32) + eb1_ref[...]
    xd_ref[...] = jnp.dot(x3, ew1b_ref[...],
                          preferred_element_type=jnp.float32)


def _f_body(h_ref, w2_ref, b2_ref, w3_ref, b3_ref, o_ref):
    h = jnp.maximum(h_ref[...], 0.0)
    h = jnp.maximum(jnp.dot(h, w2_ref[...],
                            preferred_element_type=jnp.float32) + b2_ref[...],
                    0.0)
    r = jax.nn.sigmoid(
        jnp.dot(h, w3_ref[...], preferred_element_type=jnp.float32)
        + b3_ref[...])
    o_ref[...] = r[:, 0:1]


def _full(shape):
    return pl.BlockSpec(shape, lambda i: tuple(0 for _ in shape))


def _deg_scratch():
    f32 = jnp.float32
    return [
        pltpu.VMEM((CPT, CH), jnp.int32),
        pltpu.VMEM((CH, D), f32),
        pltpu.VMEM((CH, D), f32),
        pltpu.SemaphoreType.DMA((2,)),
        pltpu.VMEM_SHARED((NPAD, D), f32),
    ]


def _conv_scratch():
    f32 = jnp.float32
    return [
        pltpu.VMEM((3, CH), jnp.int32),
        pltpu.VMEM((3, CH), jnp.int32),
        pltpu.VMEM((2, CH, D), f32),
        pltpu.SemaphoreType.DMA((3,)),
        pltpu.SemaphoreType.DMA((3,)),
        pltpu.SemaphoreType.DMA((2,)),
        pltpu.SemaphoreType.DMA((2,)),
        pltpu.VMEM_SHARED((NPAD, D), f32),
    ]


def _edge_scratch():
    f32 = jnp.float32
    return [
        pltpu.VMEM((3, CH), jnp.int32),
        pltpu.VMEM((3, CH), jnp.int32),
        pltpu.VMEM((2, CH, D), f32),
        pltpu.SemaphoreType.DMA((3,)),
        pltpu.SemaphoreType.DMA((3,)),
        pltpu.SemaphoreType.DMA((2,)),
        pltpu.SemaphoreType.DMA((2,)),
        pltpu.SemaphoreType.DMA((2,)),
    ]


def kernel(inputs, edge_index, a1_W, a1_b, a2_W, a2_b, c1_W, c1_b, c2_W, c2_b,
           c3_W, c3_b, cls_W1, cls_b1, cls_W2, cls_b2, box_W1, box_b1, lora_A,
           lora_B, e_W1, e_b1, e_W2, e_b2, e_W3, e_b3):
    f32 = jnp.float32
    pe = _pe_table()
    inp_p = jnp.pad(inputs, ((0, NPAD - N), (0, 0)))
    pads = (N + jnp.arange(EPAD - E, dtype=jnp.int32) % (NPAD - N))
    src_p = jnp.concatenate([edge_index[0], pads])
    dst_p = jnp.concatenate([edge_index[1], pads])
    srcc = src_p.reshape(NW * CPT, CH)
    dstc = dst_p.reshape(NW * CPT, CH)
    zer_d = jnp.zeros((RPT, D), f32)
    one_d = jnp.ones((CH, D), f32)

    mesh = plsc.VectorSubcoreMesh(core_axis_name="c", subcore_axis_name="s")

    deg_call = pl.kernel(
        _deg_body,
        out_type=jax.ShapeDtypeStruct((2 * NPAD, D), f32),
        mesh=mesh,
        scratch_types=_deg_scratch(),
    )
    degp = deg_call(dstc, zer_d, one_d).reshape(2, NPAD, D)

    grid = NPAD // BR
    y1, dis = pl.pallas_call(
        _a_body,
        grid=(grid,),
        in_specs=[
            pl.BlockSpec((BR, D), lambda i: (i, 0)),
            pl.BlockSpec((2, BR, D), lambda i: (0, i, 0)),
            _full((64, D)), _full((D, D)), _full((1, D)),
            _full((D, D)), _full((1, D)), _full((D, D)),
        ],
        out_specs=[pl.BlockSpec((BR, D), lambda i: (i, 0)),
                   pl.BlockSpec((BR, 1), lambda i: (i, 0))],
        out_shape=[jax.ShapeDtypeStruct((NPAD, D), f32),
                   jax.ShapeDtypeStruct((NPAD, 1), f32)],
    )(inp_p, degp, pe, a1_W, a1_b.reshape(1, D), a2_W, a2_b.reshape(1, D),
      c1_W)

    conv_call = pl.kernel(
        _conv_body,
        out_type=jax.ShapeDtypeStruct((2 * NPAD, D), f32),
        mesh=mesh,
        scratch_types=_conv_scratch(),
    )

    def conv_epilogue(accp, y, b, w):
        return pl.pallas_call(
            _c_body,
            grid=(grid,),
            in_specs=[
                pl.BlockSpec((2, BR, D), lambda i: (0, i, 0)),
                pl.BlockSpec((BR, D), lambda i: (i, 0)),
                pl.BlockSpec((BR, 1), lambda i: (i, 0)),
                _full((1, D)), _full((D, D)),
            ],
            out_specs=pl.BlockSpec((BR, D), lambda i: (i, 0)),
            out_shape=jax.ShapeDtypeStruct((NPAD, D), f32),
        )(accp, y, dis, b.reshape(1, D), w)

    accp1 = conv_call(y1, srcc, dstc, zer_d).reshape(2, NPAD, D)
    y2 = conv_epilogue(accp1, y1, c1_b, c2_W)
    accp2 = conv_call(y2, srcc, dstc, zer_d).reshape(2, NPAD, D)
    y3 = conv_epilogue(accp2, y2, c2_b, c3_W)
    accp3 = conv_call(y3, srcc, dstc, zer_d).reshape(2, NPAD, D)

    lap = jnp.pad(lora_A, ((0, 0), (0, 4)))
    lbp = jnp.pad(lora_B, ((0, 4), (0, 4)))
    x3, pred, box, xs1, xd1 = pl.pallas_call(
        _c4_body,
        grid=(grid,),
        in_specs=[
            pl.BlockSpec((2, BR, D), lambda i: (0, i, 0)),
            pl.BlockSpec((BR, D), lambda i: (i, 0)),
            pl.BlockSpec((BR, 1), lambda i: (i, 0)),
            pl.BlockSpec((BR, 4), lambda i: (i, 0)),
            _full((1, D)),
            _full((D, D // 2)), _full((1, D // 2)),
            _full((D // 2, 16)), _full((1, 16)),
            _full((D, D // 2)), _full((1, D // 2)),
            _full((D // 2, 8)), _full((8, 8)),
            _full((D, D)), _full((D, D)), _full((1, D)),
        ],
        out_specs=[pl.BlockSpec((BR, D), lambda i: (i, 0)),
                   pl.BlockSpec((BR, 16), lambda i: (i, 0)),
                   pl.BlockSpec((BR, 4), lambda i: (i, 0)),
                   pl.BlockSpec((BR, D), lambda i: (i, 0)),
                   pl.BlockSpec((BR, D), lambda i: (i, 0))],
        out_shape=[jax.ShapeDtypeStruct((NPAD, D), f32),
                   jax.ShapeDtypeStruct((NPAD, 16), f32),
                   jax.ShapeDtypeStruct((NPAD, 4), f32),
                   jax.ShapeDtypeStruct((NPAD, D), f32),
                   jax.ShapeDtypeStruct((NPAD, D), f32)],
    )(accp3, y3, dis, inp_p[:, 1:5], c3_b.reshape(1, D), cls_W1,
      cls_b1.reshape(1, D // 2), cls_W2, cls_b2.reshape(1, 16), box_W1,
      box_b1.reshape(1, D // 2), lap, lbp, e_W1[:D], e_W1[D:],
      e_b1.reshape(1, D))

    edge_call = pl.kernel(
        _edge_body,
        out_type=jax.ShapeDtypeStruct((EPAD, D), f32),
        mesh=mesh,
        scratch_types=_edge_scratch(),
    )
    h1 = edge_call(xs1, xd1, srcc, dstc)

    edge_full = pl.pallas_call(
        _f_body,
        grid=(EPAD // BE,),
        in_specs=[
            pl.BlockSpec((BE, D), lambda i: (i, 0)),
            _full((D, D // 2)), _full((1, D // 2)),
            _full((D // 2, 8)), _full((1, 8)),
        ],
        out_specs=pl.BlockSpec((BE, 1), lambda i: (i, 0)),
        out_shape=jax.ShapeDtypeStruct((EPAD, 1), f32),
        compiler_params=pltpu.CompilerParams(
            dimension_semantics=("arbitrary",)),
    )(h1, e_W2, e_b2.reshape(1, D // 2), jnp.pad(e_W3, ((0, 0), (0, 7))),
      jnp.pad(e_b3.reshape(1, 1), ((0, 0), (0, 7))))

    return (pred[:N], box[:N], edge_full[:E], x3[:N])


# speedup vs baseline: 3.7467x; 1.2578x over previous
import jax
import jax.numpy as jnp
from jax import lax
from jax.experimental import pallas as pl
from jax.experimental.pallas import tpu as pltpu
from jax.experimental.pallas import tpu_sc as plsc

N = 10000
D = 128
E = 320000
NPAD = 10240
NW = 32
CH = 128
CPT = 80
CPT0 = 80
CPT1 = 80
TCH = NW * CPT
EPT = CPT * CH
EPAD = NW * EPT
RPT = NPAD // 16
BR = 1024
BE = 4096
ZPOS = 50.0


def _pe_table():
    inv_freq = 1.0 / (55 * 10) ** (jnp.arange(0, D, 2, dtype=jnp.float32) / D)
    t = jnp.arange(0, 55, dtype=jnp.float32)[:, None]
    ang = t * inv_freq[None, :]
    pe = jnp.concatenate([jnp.sin(ang), jnp.cos(ang)], axis=1)
    return jnp.pad(pe, ((0, 64 - 55), (0, 0)))


def _deg_body(dstc_hbm, zer_hbm, one_hbm, out_hbm, di2, ones_v, rows, ssem,
              deg_sh):
    c = lax.axis_index("c")
    s = lax.axis_index("s")
    wid = c * 16 + s
    pltpu.sync_copy(zer_hbm, deg_sh.at[pl.ds(s * RPT, RPT)])
    pltpu.sync_copy(dstc_hbm.at[pl.ds(wid * CPT, CPT)], di2)
    pltpu.sync_copy(one_hbm, ones_v)
    plsc.subcore_barrier()

    def sca(j, b):
        return pltpu.make_async_copy(ones_v, deg_sh.at[di2.at[j]],
                                     ssem.at[b])

    def chunk(j, carry):
        b = lax.rem(j, 2)

        @pl.when(j >= 2)
        def _():
            sca(j - 2, b).wait()

        sca(j, b).start(add=True)
        return carry

    lax.fori_loop(0, CPT, chunk, 0)
    for j in (CPT - 2, CPT - 1):
        sca(j, j % 2).wait()
    plsc.subcore_barrier()

    bufs = [rows, ones_v]
    nrb = RPT // CH
    for k in range(nrb):
        b = bufs[k % 2]

        def wrd(kk, bb):
            return pltpu.make_async_copy(
                bb, out_hbm.at[pl.ds(c * NPAD + s * RPT + kk * CH, CH)],
                ssem.at[kk % 2])

        if k >= 2:
            wrd(k - 2, b).wait()
        pltpu.sync_copy(deg_sh.at[pl.ds(s * RPT + k * CH, CH)], b)
        wrd(k, b).start()
    for k in (nrb - 2, nrb - 1):
        pltpu.make_async_copy(
            bufs[k % 2],
            out_hbm.at[pl.ds(c * NPAD + s * RPT + k * CH, CH)],
            ssem.at[k % 2]).wait()


def _conv_body(y_hbm, srcc_hbm, dstc_hbm, zer_hbm, out_hbm, si, di, rows,
               sism, dism, gsem, ssem, acc_sh):
    c = lax.axis_index("c")
    s = lax.axis_index("s")
    cpt = jnp.where(c == 0, CPT0, CPT1)
    cb = jnp.where(c == 0, s * CPT0, 16 * CPT0 + s * CPT1)
    pltpu.sync_copy(zer_hbm, acc_sh.at[pl.ds(s * RPT, RPT)])
    plsc.subcore_barrier()

    def ils(j):
        bi = lax.rem(j, 3)
        return (pltpu.make_async_copy(srcc_hbm.at[cb + j], si.at[bi],
                                      sism.at[bi]),
                pltpu.make_async_copy(dstc_hbm.at[cb + j], di.at[bi],
                                      dism.at[bi]))

    def gat(j):
        b = lax.rem(j, 2)
        return pltpu.make_async_copy(y_hbm.at[si.at[lax.rem(j, 3)]],
                                     rows.at[b], gsem.at[b])

    def sca(j):
        b = lax.rem(j, 2)
        return pltpu.make_async_copy(rows.at[b],
                                     acc_sh.at[di.at[lax.rem(j, 3)]],
                                     ssem.at[b])

    for d in ils(0) + ils(1):
        d.start()
    for d in ils(0):
        d.wait()
    gat(0).start()

    def chunk(j, carry):
        @pl.when(j >= 1)
        def _():
            sca(j - 1).wait()

        @pl.when(j + 2 < cpt)
        def _():
            for d in ils(j + 2):
                d.start()

        @pl.when(j + 1 < cpt)
        def _():
            for d in ils(j + 1):
                d.wait()
            gat(j + 1).start()

        gat(j).wait()
        sca(j).start(add=True)
        return carry

    lax.fori_loop(0, cpt, chunk, 0)
    sca(cpt - 1).wait()
    plsc.subcore_barrier()

    def rb(k, carry):
        b = lax.rem(k, 2)
        r = s * RPT + k * CH

        @pl.when(k >= 2)
        def _():
            pltpu.make_async_copy(
                rows.at[b], out_hbm.at[pl.ds(c * NPAD + (k - 2) * CH
                                             + s * RPT, CH)],
                ssem.at[b]).wait()

        pltpu.sync_copy(acc_sh.at[pl.ds(r, CH)], rows.at[b])
        pltpu.make_async_copy(rows.at[b],
                              out_hbm.at[pl.ds(c * NPAD + r, CH)],
                              ssem.at[b]).start()
        return carry

    nrb = RPT // CH
    lax.fori_loop(0, nrb, rb, 0)
    for k in (nrb - 2, nrb - 1):
        pltpu.make_async_copy(
            rows.at[k % 2],
            out_hbm.at[pl.ds(c * NPAD + s * RPT + k * CH, CH)],
            ssem.at[k % 2]).wait()


def _edge_body(xs_hbm, xd_hbm, srcc_hbm, dstc_hbm, out_hbm, si, di, rows,
               sism, dism, gsem, asem, wsem):
    c = lax.axis_index("c")
    s = lax.axis_index("s")
    cpt = jnp.where(c == 0, CPT0, CPT1)
    cb = jnp.where(c == 0, s * CPT0, 16 * CPT0 + s * CPT1)

    def ils(j):
        bi = lax.rem(j, 3)
        return (pltpu.make_async_copy(srcc_hbm.at[cb + j], si.at[bi],
                                      sism.at[bi]),
                pltpu.make_async_copy(dstc_hbm.at[cb + j], di.at[bi],
                                      dism.at[bi]))

    def gat(j):
        b = lax.rem(j, 2)
        return pltpu.make_async_copy(xs_hbm.at[si.at[lax.rem(j, 3)]],
                                     rows.at[b], gsem.at[b])

    def gadd(j):
        b = lax.rem(j, 2)
        return pltpu.make_async_copy(xd_hbm.at[di.at[lax.rem(j, 3)]],
                                     rows.at[b], asem.at[b])

    def wr(j):
        b = lax.rem(j, 2)
        return pltpu.make_async_copy(
            rows.at[b], out_hbm.at[pl.ds((cb + j) * CH, CH)], wsem.at[b])

    for d in ils(0) + ils(1):
        d.start()
    for d in ils(0):
        d.wait()
    gat(0).start()

    def chunk(j, carry):
        @pl.when(j >= 1)
        def _():
            wr(j - 1).wait()

        @pl.when(j + 2 < cpt)
        def _():
            for d in ils(j + 2):
                d.start()

        @pl.when(j + 1 < cpt)
        def _():
            for d in ils(j + 1):
                d.wait()
            gat(j + 1).start()

        gat(j).wait()
        gadd(j).start(add=True)
        gadd(j).wait()
        wr(j).start()
        return carry

    lax.fori_loop(0, cpt, chunk, 0)
    wr(cpt - 1).wait()


def _a_body(inp_ref, degp_ref, pe_ref, a1w_ref, a1b_ref, a2w_ref, a2b_ref,
            c1w_ref, y1_ref, dis_ref):
    xb = inp_ref[...]
    t = jnp.maximum(jnp.dot(xb, a1w_ref[...],
                            preferred_element_type=jnp.float32) + a1b_ref[...],
                    0.0)
    f = jnp.dot(t, a2w_ref[...], preferred_element_type=jnp.float32) \
        + a2b_ref[...]
    pos = (xb[:, 0:1] * ZPOS).astype(jnp.int32)
    iot = lax.broadcasted_iota(jnp.int32, (1, 64), 1)
    oh = (pos == iot).astype(jnp.float32)
    x0 = f + jnp.dot(oh, pe_ref[...], preferred_element_type=jnp.float32)
    dp = degp_ref[...]
    deg = dp[0, :, 0:1] + dp[1, :, 0:1] + 1.0
    dis = lax.rsqrt(deg)
    dis_ref[...] = dis
    y1_ref[...] = dis * jnp.dot(x0, c1w_ref[...],
                                preferred_element_type=jnp.float32)


def _c_body(accp_ref, y_ref, dis_ref, b_ref, w_ref, yout_ref):
    ap = accp_ref[...]
    dis = dis_ref[...]
    x = jnp.maximum(dis * (ap[0] + ap[1] + y_ref[...]) + b_ref[...], 0.0)
    yout_ref[...] = dis * jnp.dot(x, w_ref[...],
                                  preferred_element_type=jnp.float32)


def _c4_body(accp_ref, y_ref, dis_ref, inp4_ref, c3b_ref, clsw1_ref,
             clsb1_ref, clsw2_ref, clsb2_ref, boxw1_ref, boxb1_ref, la_ref,
             lb_ref, ew1a_ref, ew1b_ref, eb1_ref, x3_ref, pred_ref, box_ref,
             xs_ref, xd_ref):
    ap = accp_ref[...]
    dis = dis_ref[...]
    x3 = dis * (ap[0] + ap[1] + y_ref[...]) + c3b_ref[...]
    x3_ref[...] = x3
    p = jnp.maximum(jnp.dot(x3, clsw1_ref[...],
                            preferred_element_type=jnp.float32)
                    + clsb1_ref[...], 0.0)
    pred_ref[...] = jnp.dot(p, clsw2_ref[...],
                            preferred_element_type=jnp.float32) + clsb2_ref[...]
    h = jnp.maximum(jnp.dot(x3, boxw1_ref[...],
                            preferred_element_type=jnp.float32)
                    + boxb1_ref[...], 0.0)
    ha = jnp.dot(h, la_ref[...], preferred_element_type=jnp.float32)
    hb = jnp.dot(ha, lb_ref[...], preferred_element_type=jnp.float32)
    box_ref[...] = jnp.tanh(hb[:, 0:4]) + inp4_ref[...]
    xs_ref[...] = jnp.dot(x3, ew1a_ref[...],
                          preferred_element_type=jnp.float32) + eb1_ref[...]
    xd_ref[...] = jnp.dot(x3, ew1b_ref[...],
                          preferred_element_type=jnp.float32)


def _f_body(h_ref, w2_ref, b2_ref, w3t_ref, b3_ref, o_ref):
    h = jnp.maximum(h_ref[...], 0.0)
    h = jnp.maximum(jnp.dot(h, w2_ref[...],
                            preferred_element_type=jnp.float32) + b2_ref[...],
                    0.0)
    rt = lax.dot_general(w3t_ref[...], h, (((1,), (1,)), ((), ())),
                         preferred_element_type=jnp.float32)
    o_ref[...] = jax.nn.sigmoid(rt[0:1, :] + b3_ref[...]).reshape(1, 1, -1)


def _full(shape):
    return pl.BlockSpec(shape, lambda i: tuple(0 for _ in shape))


def _deg_scratch():
    f32 = jnp.float32
    return [
        pltpu.VMEM((CPT, CH), jnp.int32),
        pltpu.VMEM((CH, D), f32),
        pltpu.VMEM((CH, D), f32),
        pltpu.SemaphoreType.DMA((2,)),
        pltpu.VMEM_SHARED((NPAD, D), f32),
    ]


def _conv_scratch():
    f32 = jnp.float32
    return [
        pltpu.VMEM((3, CH), jnp.int32),
        pltpu.VMEM((3, CH), jnp.int32),
        pltpu.VMEM((2, CH, D), f32),
        pltpu.SemaphoreType.DMA((3,)),
        pltpu.SemaphoreType.DMA((3,)),
        pltpu.SemaphoreType.DMA((2,)),
        pltpu.SemaphoreType.DMA((2,)),
        pltpu.VMEM_SHARED((NPAD, D), f32),
    ]


def _edge_scratch():
    f32 = jnp.float32
    return [
        pltpu.VMEM((3, CH), jnp.int32),
        pltpu.VMEM((3, CH), jnp.int32),
        pltpu.VMEM((2, CH, D), f32),
        pltpu.SemaphoreType.DMA((3,)),
        pltpu.SemaphoreType.DMA((3,)),
        pltpu.SemaphoreType.DMA((2,)),
        pltpu.SemaphoreType.DMA((2,)),
        pltpu.SemaphoreType.DMA((2,)),
    ]


def kernel(inputs, edge_index, a1_W, a1_b, a2_W, a2_b, c1_W, c1_b, c2_W, c2_b,
           c3_W, c3_b, cls_W1, cls_b1, cls_W2, cls_b2, box_W1, box_b1, lora_A,
           lora_B, e_W1, e_b1, e_W2, e_b2, e_W3, e_b3):
    f32 = jnp.float32
    pe = _pe_table()
    inp_p = jnp.pad(inputs, ((0, NPAD - N), (0, 0)))
    pads = (N + jnp.arange(EPAD - E, dtype=jnp.int32) % (NPAD - N))
    src_p = jnp.concatenate([edge_index[0], pads])
    dst_p = jnp.concatenate([edge_index[1], pads])
    srcc = src_p.reshape(NW * CPT, CH)
    dstc = dst_p.reshape(NW * CPT, CH)
    zer_d = jnp.zeros((RPT, D), f32)
    one_d = jnp.ones((CH, D), f32)

    mesh = plsc.VectorSubcoreMesh(core_axis_name="c", subcore_axis_name="s")

    deg_call = pl.kernel(
        _deg_body,
        out_type=jax.ShapeDtypeStruct((2 * NPAD, D), f32),
        mesh=mesh,
        scratch_types=_deg_scratch(),
    )
    degp = deg_call(dstc, zer_d, one_d).reshape(2, NPAD, D)

    grid = NPAD // BR
    y1, dis = pl.pallas_call(
        _a_body,
        grid=(grid,),
        in_specs=[
            pl.BlockSpec((BR, D), lambda i: (i, 0)),
            pl.BlockSpec((2, BR, D), lambda i: (0, i, 0)),
            _full((64, D)), _full((D, D)), _full((1, D)),
            _full((D, D)), _full((1, D)), _full((D, D)),
        ],
        out_specs=[pl.BlockSpec((BR, D), lambda i: (i, 0)),
                   pl.BlockSpec((BR, 1), lambda i: (i, 0))],
        out_shape=[jax.ShapeDtypeStruct((NPAD, D), f32),
                   jax.ShapeDtypeStruct((NPAD, 1), f32)],
    )(inp_p, degp, pe, a1_W, a1_b.reshape(1, D), a2_W, a2_b.reshape(1, D),
      c1_W)

    conv_call = pl.kernel(
        _conv_body,
        out_type=jax.ShapeDtypeStruct((2 * NPAD, D), f32),
        mesh=mesh,
        scratch_types=_conv_scratch(),
    )

    def conv_epilogue(accp, y, b, w):
        return pl.pallas_call(
            _c_body,
            grid=(grid,),
            in_specs=[
                pl.BlockSpec((2, BR, D), lambda i: (0, i, 0)),
                pl.BlockSpec((BR, D), lambda i: (i, 0)),
                pl.BlockSpec((BR, 1), lambda i: (i, 0)),
                _full((1, D)), _full((D, D)),
            ],
            out_specs=pl.BlockSpec((BR, D), lambda i: (i, 0)),
            out_shape=jax.ShapeDtypeStruct((NPAD, D), f32),
        )(accp, y, dis, b.reshape(1, D), w)

    accp1 = conv_call(y1, srcc, dstc, zer_d).reshape(2, NPAD, D)
    y2 = conv_epilogue(accp1, y1, c1_b, c2_W)
    accp2 = conv_call(y2, srcc, dstc, zer_d).reshape(2, NPAD, D)
    y3 = conv_epilogue(accp2, y2, c2_b, c3_W)
    accp3 = conv_call(y3, srcc, dstc, zer_d).reshape(2, NPAD, D)

    lap = jnp.pad(lora_A, ((0, 0), (0, 4)))
    lbp = jnp.pad(lora_B, ((0, 4), (0, 4)))
    x3, pred, box, xs1, xd1 = pl.pallas_call(
        _c4_body,
        grid=(grid,),
        in_specs=[
            pl.BlockSpec((2, BR, D), lambda i: (0, i, 0)),
            pl.BlockSpec((BR, D), lambda i: (i, 0)),
            pl.BlockSpec((BR, 1), lambda i: (i, 0)),
            pl.BlockSpec((BR, 4), lambda i: (i, 0)),
            _full((1, D)),
            _full((D, D // 2)), _full((1, D // 2)),
            _full((D // 2, 16)), _full((1, 16)),
            _full((D, D // 2)), _full((1, D // 2)),
            _full((D // 2, 8)), _full((8, 8)),
            _full((D, D)), _full((D, D)), _full((1, D)),
        ],
        out_specs=[pl.BlockSpec((BR, D), lambda i: (i, 0)),
                   pl.BlockSpec((BR, 16), lambda i: (i, 0)),
                   pl.BlockSpec((BR, 4), lambda i: (i, 0)),
                   pl.BlockSpec((BR, D), lambda i: (i, 0)),
                   pl.BlockSpec((BR, D), lambda i: (i, 0))],
        out_shape=[jax.ShapeDtypeStruct((NPAD, D), f32),
                   jax.ShapeDtypeStruct((NPAD, 16), f32),
                   jax.ShapeDtypeStruct((NPAD, 4), f32),
                   jax.ShapeDtypeStruct((NPAD, D), f32),
                   jax.ShapeDtypeStruct((NPAD, D), f32)],
    )(accp3, y3, dis, inp_p[:, 1:5], c3_b.reshape(1, D), cls_W1,
      cls_b1.reshape(1, D // 2), cls_W2, cls_b2.reshape(1, 16), box_W1,
      box_b1.reshape(1, D // 2), lap, lbp, e_W1[:D], e_W1[D:],
      e_b1.reshape(1, D))

    edge_call = pl.kernel(
        _edge_body,
        out_type=jax.ShapeDtypeStruct((EPAD, D), f32),
        mesh=mesh,
        scratch_types=_edge_scratch(),
    )
    h1 = edge_call(xs1, xd1, srcc, dstc)

    edge_full = pl.pallas_call(
        _f_body,
        grid=(EPAD // BE,),
        in_specs=[
            pl.BlockSpec((BE, D), lambda i: (i, 0)),
            _full((D, D // 2)), _full((1, D // 2)),
            _full((8, D // 2)), _full((1, 1)),
        ],
        out_specs=pl.BlockSpec((1, 1, BE), lambda i: (i, 0, 0)),
        out_shape=jax.ShapeDtypeStruct((EPAD // BE, 1, BE), f32),
        compiler_params=pltpu.CompilerParams(
            dimension_semantics=("arbitrary",)),
    )(h1, e_W2, e_b2.reshape(1, D // 2),
      jnp.pad(e_W3.T, ((0, 7), (0, 0))), e_b3.reshape(1, 1))

    edge = edge_full.reshape(EPAD)[:E].reshape(E, 1)
    return (pred[:N], box[:N], edge, x3[:N])


# speedup vs baseline: 3.9369x; 1.0508x over previous
import jax
import jax.numpy as jnp
from jax import lax
from jax.experimental import pallas as pl
from jax.experimental.pallas import tpu as pltpu
from jax.experimental.pallas import tpu_sc as plsc

N = 10000
D = 128
E = 320000
NPAD = 10240
NW = 32
CH = 128
CPT = 80
CPT0 = 80
CPT1 = 80
TCH = NW * CPT
EPT = CPT * CH
EPAD = NW * EPT
RPT = NPAD // 16
BR = 1024
BE = 4096
ZPOS = 50.0


def _pe_table():
    inv_freq = 1.0 / (55 * 10) ** (jnp.arange(0, D, 2, dtype=jnp.float32) / D)
    t = jnp.arange(0, 55, dtype=jnp.float32)[:, None]
    ang = t * inv_freq[None, :]
    pe = jnp.concatenate([jnp.sin(ang), jnp.cos(ang)], axis=1)
    return jnp.pad(pe, ((0, 64 - 55), (0, 0)))


def _deg_body(dstc_hbm, zer_hbm, one_hbm, out_hbm, di2, ones_v, rows, ssem,
              deg_sh):
    c = lax.axis_index("c")
    s = lax.axis_index("s")
    wid = c * 16 + s
    pltpu.sync_copy(zer_hbm, deg_sh.at[pl.ds(s * RPT, RPT)])
    pltpu.sync_copy(dstc_hbm.at[pl.ds(wid * CPT, CPT)], di2)
    pltpu.sync_copy(one_hbm, ones_v)
    plsc.subcore_barrier()

    def sca(j, b):
        return pltpu.make_async_copy(ones_v, deg_sh.at[di2.at[j]],
                                     ssem.at[b])

    def chunk(j, carry):
        b = lax.rem(j, 2)

        @pl.when(j >= 2)
        def _():
            sca(j - 2, b).wait()

        sca(j, b).start(add=True)
        return carry

    lax.fori_loop(0, CPT, chunk, 0)
    for j in (CPT - 2, CPT - 1):
        sca(j, j % 2).wait()
    plsc.subcore_barrier()

    bufs = [rows, ones_v]
    nrb = RPT // CH
    for k in range(nrb):
        b = bufs[k % 2]

        def wrd(kk, bb):
            return pltpu.make_async_copy(
                bb, out_hbm.at[pl.ds(c * NPAD + s * RPT + kk * CH, CH)],
                ssem.at[kk % 2])

        if k >= 2:
            wrd(k - 2, b).wait()
        pltpu.sync_copy(deg_sh.at[pl.ds(s * RPT + k * CH, CH)], b)
        wrd(k, b).start()
    for k in (nrb - 2, nrb - 1):
        pltpu.make_async_copy(
            bufs[k % 2],
            out_hbm.at[pl.ds(c * NPAD + s * RPT + k * CH, CH)],
            ssem.at[k % 2]).wait()


def _conv_body(y_hbm, srcc_hbm, dstc_hbm, zer_hbm, out_hbm, si, di, rows,
               sism, dism, gsem, ssem, acc_sh):
    c = lax.axis_index("c")
    s = lax.axis_index("s")
    cpt = jnp.where(c == 0, CPT0, CPT1)
    cb = jnp.where(c == 0, s * CPT0, 16 * CPT0 + s * CPT1)
    pltpu.sync_copy(zer_hbm, acc_sh.at[pl.ds(s * RPT, RPT)])
    plsc.subcore_barrier()

    def ils(j):
        bi = lax.rem(j, 3)
        return (pltpu.make_async_copy(srcc_hbm.at[cb + j], si.at[bi],
                                      sism.at[bi]),
                pltpu.make_async_copy(dstc_hbm.at[cb + j], di.at[bi],
                                      dism.at[bi]))

    def gat(j):
        b = lax.rem(j, 2)
        return pltpu.make_async_copy(y_hbm.at[si.at[lax.rem(j, 3)]],
                                     rows.at[b], gsem.at[b])

    def sca(j):
        b = lax.rem(j, 2)
        return pltpu.make_async_copy(rows.at[b],
                                     acc_sh.at[di.at[lax.rem(j, 3)]],
                                     ssem.at[b])

    for d in ils(0) + ils(1):
        d.start()
    for d in ils(0):
        d.wait()
    gat(0).start()

    def chunk(j, carry):
        @pl.when(j >= 1)
        def _():
            sca(j - 1).wait()

        @pl.when(j + 2 < cpt)
        def _():
            for d in ils(j + 2):
                d.start()

        @pl.when(j + 1 < cpt)
        def _():
            for d in ils(j + 1):
                d.wait()
            gat(j + 1).start()

        gat(j).wait()
        sca(j).start(add=True)
        return carry

    lax.fori_loop(0, cpt, chunk, 0)
    sca(cpt - 1).wait()
    plsc.subcore_barrier()

    def rb(k, carry):
        b = lax.rem(k, 2)
        r = s * RPT + k * CH

        @pl.when(k >= 2)
        def _():
            pltpu.make_async_copy(
                rows.at[b], out_hbm.at[pl.ds(c * NPAD + (k - 2) * CH
                                             + s * RPT, CH)],
                ssem.at[b]).wait()

        pltpu.sync_copy(acc_sh.at[pl.ds(r, CH)], rows.at[b])
        pltpu.make_async_copy(rows.at[b],
                              out_hbm.at[pl.ds(c * NPAD + r, CH)],
                              ssem.at[b]).start()
        return carry

    nrb = RPT // CH
    lax.fori_loop(0, nrb, rb, 0)
    for k in (nrb - 2, nrb - 1):
        pltpu.make_async_copy(
            rows.at[k % 2],
            out_hbm.at[pl.ds(c * NPAD + s * RPT + k * CH, CH)],
            ssem.at[k % 2]).wait()


def _edge_body(xs_hbm, xd_hbm, srcc_hbm, dstc_hbm, out_hbm, si, di, rows,
               sism, dism, gsem, asem, wsem):
    c = lax.axis_index("c")
    s = lax.axis_index("s")
    cpt = jnp.where(c == 0, CPT0, CPT1)
    cb = jnp.where(c == 0, s * CPT0, 16 * CPT0 + s * CPT1)

    def ils(j):
        bi = lax.rem(j, 5)
        return (pltpu.make_async_copy(srcc_hbm.at[cb + j], si.at[bi],
                                      sism.at[bi]),
                pltpu.make_async_copy(dstc_hbm.at[cb + j], di.at[bi],
                                      dism.at[bi]))

    def gat(j):
        b = lax.rem(j, 4)
        return pltpu.make_async_copy(xs_hbm.at[si.at[lax.rem(j, 5)]],
                                     rows.at[b], gsem.at[b])

    def gadd(j):
        b = lax.rem(j, 4)
        return pltpu.make_async_copy(xd_hbm.at[di.at[lax.rem(j, 5)]],
                                     rows.at[b], asem.at[b])

    def wr(j):
        b = lax.rem(j, 4)
        return pltpu.make_async_copy(
            rows.at[b], out_hbm.at[pl.ds((cb + j) * CH, CH)], wsem.at[b])

    for j0 in (0, 1, 2):
        for d in ils(j0):
            d.start()
    for j0 in (0, 1):
        for d in ils(j0):
            d.wait()
        gat(j0).start()

    def chunk(j, carry):
        @pl.when(j >= 2)
        def _():
            wr(j - 2).wait()

        @pl.when(j + 3 < cpt)
        def _():
            for d in ils(j + 3):
                d.start()

        @pl.when(j + 2 < cpt)
        def _():
            for d in ils(j + 2):
                d.wait()
            gat(j + 2).start()

        gat(j).wait()
        gadd(j).start(add=True)

        @pl.when(j >= 1)
        def _():
            gadd(j - 1).wait()
            wr(j - 1).start()

        return carry

    lax.fori_loop(0, cpt, chunk, 0)
    gadd(cpt - 1).wait()
    wr(cpt - 1).start()
    wr(cpt - 2).wait()
    wr(cpt - 1).wait()


def _a_body(inp_ref, degp_ref, pe_ref, a1w_ref, a1b_ref, a2w_ref, a2b_ref,
            c1w_ref, y1_ref, dis_ref):
    xb = inp_ref[...]
    t = jnp.maximum(jnp.dot(xb, a1w_ref[...],
                            preferred_element_type=jnp.float32) + a1b_ref[...],
                    0.0)
    f = jnp.dot(t, a2w_ref[...], preferred_element_type=jnp.float32) \
        + a2b_ref[...]
    pos = (xb[:, 0:1] * ZPOS).astype(jnp.int32)
    iot = lax.broadcasted_iota(jnp.int32, (1, 64), 1)
    oh = (pos == iot).astype(jnp.float32)
    x0 = f + jnp.dot(oh, pe_ref[...], preferred_element_type=jnp.float32)
    dp = degp_ref[...]
    deg = dp[0, :, 0:1] + dp[1, :, 0:1] + 1.0
    dis = lax.rsqrt(deg)
    dis_ref[...] = dis
    y1_ref[...] = dis * jnp.dot(x0, c1w_ref[...],
                                preferred_element_type=jnp.float32)


def _c_body(accp_ref, y_ref, dis_ref, b_ref, w_ref, yout_ref):
    ap = accp_ref[...]
    dis = dis_ref[...]
    x = jnp.maximum(dis * (ap[0] + ap[1] + y_ref[...]) + b_ref[...], 0.0)
    yout_ref[...] = dis * jnp.dot(x, w_ref[...],
                                  preferred_element_type=jnp.float32)


def _c4_body(accp_ref, y_ref, dis_ref, inp4_ref, c3b_ref, clsw1_ref,
             clsb1_ref, clsw2_ref, clsb2_ref, boxw1_ref, boxb1_ref, la_ref,
             lb_ref, ew1a_ref, ew1b_ref, eb1_ref, x3_ref, pred_ref, box_ref,
             xs_ref, xd_ref):
    ap = accp_ref[...]
    dis = dis_ref[...]
    x3 = dis * (ap[0] + ap[1] + y_ref[...]) + c3b_ref[...]
    x3_ref[...] = x3
    p = jnp.maximum(jnp.dot(x3, clsw1_ref[...],
                            preferred_element_type=jnp.float32)
                    + clsb1_ref[...], 0.0)
    pred_ref[...] = jnp.dot(p, clsw2_ref[...],
                            preferred_element_type=jnp.float32) + clsb2_ref[...]
    h = jnp.maximum(jnp.dot(x3, boxw1_ref[...],
                            preferred_element_type=jnp.float32)
                    + boxb1_ref[...], 0.0)
    ha = jnp.dot(h, la_ref[...], preferred_element_type=jnp.float32)
    hb = jnp.dot(ha, lb_ref[...], preferred_element_type=jnp.float32)
    box_ref[...] = jnp.tanh(hb[:, 0:4]) + inp4_ref[...]
    xs_ref[...] = jnp.dot(x3, ew1a_ref[...],
                          preferred_element_type=jnp.float32) + eb1_ref[...]
    xd_ref[...] = jnp.dot(x3, ew1b_ref[...],
                          preferred_element_type=jnp.float32)


def _f_body(h_ref, w2_ref, b2_ref, w3t_ref, b3_ref, o_ref):
    h = jnp.maximum(h_ref[...], 0.0)
    h = jnp.maximum(jnp.dot(h, w2_ref[...],
                            preferred_element_type=jnp.float32) + b2_ref[...],
                    0.0)
    rt = lax.dot_general(w3t_ref[...], h, (((1,), (1,)), ((), ())),
                         preferred_element_type=jnp.float32)
    o_ref[...] = jax.nn.sigmoid(rt[0:1, :] + b3_ref[...]).reshape(1, 1, -1)


def _full(shape):
    return pl.BlockSpec(shape, lambda i: tuple(0 for _ in shape))


def _deg_scratch():
    f32 = jnp.float32
    return [
        pltpu.VMEM((CPT, CH), jnp.int32),
        pltpu.VMEM((CH, D), f32),
        pltpu.VMEM((CH, D), f32),
        pltpu.SemaphoreType.DMA((2,)),
        pltpu.VMEM_SHARED((NPAD, D), f32),
    ]


def _conv_scratch():
    f32 = jnp.float32
    return [
        pltpu.VMEM((3, CH), jnp.int32),
        pltpu.VMEM((3, CH), jnp.int32),
        pltpu.VMEM((2, CH, D), f32),
        pltpu.SemaphoreType.DMA((3,)),
        pltpu.SemaphoreType.DMA((3,)),
        pltpu.SemaphoreType.DMA((2,)),
        pltpu.SemaphoreType.DMA((2,)),
        pltpu.VMEM_SHARED((NPAD, D), f32),
    ]


def _edge_scratch():
    f32 = jnp.float32
    return [
        pltpu.VMEM((5, CH), jnp.int32),
        pltpu.VMEM((5, CH), jnp.int32),
        pltpu.VMEM((4, CH, D), f32),
        pltpu.SemaphoreType.DMA((5,)),
        pltpu.SemaphoreType.DMA((5,)),
        pltpu.SemaphoreType.DMA((4,)),
        pltpu.SemaphoreType.DMA((4,)),
        pltpu.SemaphoreType.DMA((4,)),
    ]


def kernel(inputs, edge_index, a1_W, a1_b, a2_W, a2_b, c1_W, c1_b, c2_W, c2_b,
           c3_W, c3_b, cls_W1, cls_b1, cls_W2, cls_b2, box_W1, box_b1, lora_A,
           lora_B, e_W1, e_b1, e_W2, e_b2, e_W3, e_b3):
    f32 = jnp.float32
    pe = _pe_table()
    inp_p = jnp.pad(inputs, ((0, NPAD - N), (0, 0)))
    pads = (N + jnp.arange(EPAD - E, dtype=jnp.int32) % (NPAD - N))
    src_p = jnp.concatenate([edge_index[0], pads])
    dst_p = jnp.concatenate([edge_index[1], pads])
    srcc = src_p.reshape(NW * CPT, CH)
    dstc = dst_p.reshape(NW * CPT, CH)
    zer_d = jnp.zeros((RPT, D), f32)
    one_d = jnp.ones((CH, D), f32)

    mesh = plsc.VectorSubcoreMesh(core_axis_name="c", subcore_axis_name="s")

    deg_call = pl.kernel(
        _deg_body,
        out_type=jax.ShapeDtypeStruct((2 * NPAD, D), f32),
        mesh=mesh,
        scratch_types=_deg_scratch(),
    )
    degp = deg_call(dstc, zer_d, one_d).reshape(2, NPAD, D)

    grid = NPAD // BR
    y1, dis = pl.pallas_call(
        _a_body,
        grid=(grid,),
        in_specs=[
            pl.BlockSpec((BR, D), lambda i: (i, 0)),
            pl.BlockSpec((2, BR, D), lambda i: (0, i, 0)),
            _full((64, D)), _full((D, D)), _full((1, D)),
            _full((D, D)), _full((1, D)), _full((D, D)),
        ],
        out_specs=[pl.BlockSpec((BR, D), lambda i: (i, 0)),
                   pl.BlockSpec((BR, 1), lambda i: (i, 0))],
        out_shape=[jax.ShapeDtypeStruct((NPAD, D), f32),
                   jax.ShapeDtypeStruct((NPAD, 1), f32)],
    )(inp_p, degp, pe, a1_W, a1_b.reshape(1, D), a2_W, a2_b.reshape(1, D),
      c1_W)

    conv_call = pl.kernel(
        _conv_body,
        out_type=jax.ShapeDtypeStruct((2 * NPAD, D), f32),
        mesh=mesh,
        scratch_types=_conv_scratch(),
    )

    def conv_epilogue(accp, y, b, w):
        return pl.pallas_call(
            _c_body,
            grid=(grid,),
            in_specs=[
                pl.BlockSpec((2, BR, D), lambda i: (0, i, 0)),
                pl.BlockSpec((BR, D), lambda i: (i, 0)),
                pl.BlockSpec((BR, 1), lambda i: (i, 0)),
                _full((1, D)), _full((D, D)),
            ],
            out_specs=pl.BlockSpec((BR, D), lambda i: (i, 0)),
            out_shape=jax.ShapeDtypeStruct((NPAD, D), f32),
        )(accp, y, dis, b.reshape(1, D), w)

    accp1 = conv_call(y1, srcc, dstc, zer_d).reshape(2, NPAD, D)
    y2 = conv_epilogue(accp1, y1, c1_b, c2_W)
    accp2 = conv_call(y2, srcc, dstc, zer_d).reshape(2, NPAD, D)
    y3 = conv_epilogue(accp2, y2, c2_b, c3_W)
    accp3 = conv_call(y3, srcc, dstc, zer_d).reshape(2, NPAD, D)

    lap = jnp.pad(lora_A, ((0, 0), (0, 4)))
    lbp = jnp.pad(lora_B, ((0, 4), (0, 4)))
    x3, pred, box, xs1, xd1 = pl.pallas_call(
        _c4_body,
        grid=(grid,),
        in_specs=[
            pl.BlockSpec((2, BR, D), lambda i: (0, i, 0)),
            pl.BlockSpec((BR, D), lambda i: (i, 0)),
            pl.BlockSpec((BR, 1), lambda i: (i, 0)),
            pl.BlockSpec((BR, 4), lambda i: (i, 0)),
            _full((1, D)),
            _full((D, D // 2)), _full((1, D // 2)),
            _full((D // 2, 16)), _full((1, 16)),
            _full((D, D // 2)), _full((1, D // 2)),
            _full((D // 2, 8)), _full((8, 8)),
            _full((D, D)), _full((D, D)), _full((1, D)),
        ],
        out_specs=[pl.BlockSpec((BR, D), lambda i: (i, 0)),
                   pl.BlockSpec((BR, 16), lambda i: (i, 0)),
                   pl.BlockSpec((BR, 4), lambda i: (i, 0)),
                   pl.BlockSpec((BR, D), lambda i: (i, 0)),
                   pl.BlockSpec((BR, D), lambda i: (i, 0))],
        out_shape=[jax.ShapeDtypeStruct((NPAD, D), f32),
                   jax.ShapeDtypeStruct((NPAD, 16), f32),
                   jax.ShapeDtypeStruct((NPAD, 4), f32),
                   jax.ShapeDtypeStruct((NPAD, D), f32),
                   jax.ShapeDtypeStruct((NPAD, D), f32)],
    )(accp3, y3, dis, inp_p[:, 1:5], c3_b.reshape(1, D), cls_W1,
      cls_b1.reshape(1, D // 2), cls_W2, cls_b2.reshape(1, 16), box_W1,
      box_b1.reshape(1, D // 2), lap, lbp, e_W1[:D], e_W1[D:],
      e_b1.reshape(1, D))

    edge_call = pl.kernel(
        _edge_body,
        out_type=jax.ShapeDtypeStruct((EPAD, D), f32),
        mesh=mesh,
        scratch_types=_edge_scratch(),
    )
    h1 = edge_call(xs1, xd1, srcc, dstc)

    edge_full = pl.pallas_call(
        _f_body,
        grid=(EPAD // BE,),
        in_specs=[
            pl.BlockSpec((BE, D), lambda i: (i, 0)),
            _full((D, D // 2)), _full((1, D // 2)),
            _full((8, D // 2)), _full((1, 1)),
        ],
        out_specs=pl.BlockSpec((1, 1, BE), lambda i: (i, 0, 0)),
        out_shape=jax.ShapeDtypeStruct((EPAD // BE, 1, BE), f32),
        compiler_params=pltpu.CompilerParams(
            dimension_semantics=("arbitrary",)),
    )(h1, e_W2, e_b2.reshape(1, D // 2),
      jnp.pad(e_W3.T, ((0, 7), (0, 0))), e_b3.reshape(1, 1))

    edge = edge_full.reshape(EPAD)[:E].reshape(E, 1)
    return (pred[:N], box[:N], edge, x3[:N])
